# baseline (speedup 1.0000x reference)
import functools

import jax
import jax.numpy as jnp
from jax import lax
from jax.experimental import pallas as pl
from jax.experimental.pallas import tpu as pltpu
from jax.experimental.pallas import tpu_sc as plsc

_EPS = 1e-5
_LB = 2048
_RBP = 2048
_CH = 128
_CPG = 5
_GRP = _CH * _CPG
_NBUF = 3
_PREC = lax.Precision.DEFAULT


def _gather_rows(table, idx_flat):
    G = idx_flat.shape[0]
    info = plsc.get_sparse_core_info()
    NC, NS = info.num_cores, info.num_subcores
    NW = NC * NS
    per_w = G // NW
    n_groups = per_w // _GRP
    assert per_w % _GRP == 0 and G % NW == 0
    D = table.shape[1]
    mesh = plsc.VectorSubcoreMesh(core_axis_name="c", subcore_axis_name="s")

    @functools.partial(
        pl.kernel,
        mesh=mesh,
        compiler_params=pltpu.CompilerParams(use_tc_tiling_on_sc=False),
        cost_estimate=pl.CostEstimate(
            flops=0, bytes_accessed=int(G * D * 4 * 2), transcendentals=0),
        out_type=jax.ShapeDtypeStruct((G, D), jnp.float32),
        scratch_types=[
            pltpu.VMEM((_GRP,), jnp.int32),
            pltpu.VMEM((_GRP,), jnp.int32),
            pltpu.VMEM((_GRP,), jnp.int32),
            pltpu.VMEM((_GRP, D), jnp.float32),
            pltpu.VMEM((_GRP, D), jnp.float32),
            pltpu.VMEM((_GRP, D), jnp.float32),
            pltpu.SemaphoreType.DMA,
            pltpu.SemaphoreType.DMA,
            pltpu.SemaphoreType.DMA,
            pltpu.SemaphoreType.DMA,
            pltpu.SemaphoreType.DMA,
            pltpu.SemaphoreType.DMA,
        ],
    )
    def k(table_hbm, idx_hbm, out_hbm, i0, i1, i2, r0, r1, r2,
          gs0, gs1, gs2, ss0, ss1, ss2):
        idxs = [i0, i1, i2]
        rows = [r0, r1, r2]
        gsem = [gs0, gs1, gs2]
        ssem = [ss0, ss1, ss2]
        wid = lax.axis_index("s") * NC + lax.axis_index("c")
        base_w = wid * per_w
        gcopies = [None] * _NBUF
        stores = [None] * _NBUF
        for g in range(n_groups):
            b = g % _NBUF
            if g >= _NBUF:
                stores[b].wait()
            pltpu.sync_copy(idx_hbm.at[pl.ds(base_w + g * _GRP, _GRP)],
                            idxs[b])
            cs = []
            for j in range(_CPG):
                cs.append(pltpu.async_copy(
                    table_hbm.at[idxs[b].at[pl.ds(j * _CH, _CH)]],
                    rows[b].at[pl.ds(j * _CH, _CH)], gsem[b]))
            gcopies[b] = cs
            if g >= 1:
                pb = (g - 1) % _NBUF
                for c in gcopies[pb]:
                    c.wait()
                stores[pb] = pltpu.async_copy(
                    rows[pb],
                    out_hbm.at[pl.ds(base_w + (g - 1) * _GRP, _GRP)],
                    ssem[pb])
        lb = (n_groups - 1) % _NBUF
        for c in gcopies[lb]:
            c.wait()
        stores[lb] = pltpu.async_copy(
            rows[lb],
            out_hbm.at[pl.ds(base_w + (n_groups - 1) * _GRP, _GRP)],
            ssem[lb])
        for b in range(_NBUF):
            stores[b].wait()

    return k(table, idx_flat)


def _conv_stats_cn(sp, st, tex, w_sp, w_st, w_tx, n):
    nb = -(-n // _LB)
    co = w_sp.shape[0]

    def body(sp_ref, st_ref, tex_ref, wa_ref, wb_ref, wc_ref,
             y_ref, s1_ref, s2_ref):
        i = pl.program_id(0)
        dn = (((1,), (0,)), ((), ()))
        y = lax.dot_general(wa_ref[...], sp_ref[...], dn,
                            preferred_element_type=jnp.float32,
                            precision=_PREC)
        y += lax.dot_general(wb_ref[...], st_ref[...], dn,
                             preferred_element_type=jnp.float32,
                             precision=_PREC)
        y += lax.dot_general(wc_ref[...], tex_ref[...], dn,
                             preferred_element_type=jnp.float32,
                             precision=_PREC)
        y_ref[...] = y.astype(jnp.bfloat16)
        ids = i * _LB + lax.broadcasted_iota(jnp.int32, y.shape, 1)
        ym = jnp.where(ids < n, y, 0.0)

        @pl.when(i == 0)
        def _():
            s1_ref[...] = jnp.zeros_like(s1_ref)
            s2_ref[...] = jnp.zeros_like(s2_ref)

        s1_ref[...] += jnp.sum(ym, axis=1, keepdims=True)
        s2_ref[...] += jnp.sum(ym * ym, axis=1, keepdims=True)

    c_sp, c_st, c_tx = sp.shape[0], st.shape[0], tex.shape[0]
    return pl.pallas_call(
        body,
        grid=(nb,),
        in_specs=[
            pl.BlockSpec((c_sp, _LB), lambda i: (0, i)),
            pl.BlockSpec((c_st, _LB), lambda i: (0, i)),
            pl.BlockSpec((c_tx, _LB), lambda i: (0, i)),
            pl.BlockSpec((co, c_sp), lambda i: (0, 0)),
            pl.BlockSpec((co, c_st), lambda i: (0, 0)),
            pl.BlockSpec((co, c_tx), lambda i: (0, 0)),
        ],
        out_specs=[
            pl.BlockSpec((co, _LB), lambda i: (0, i)),
            pl.BlockSpec((co, 1), lambda i: (0, 0)),
            pl.BlockSpec((co, 1), lambda i: (0, 0)),
        ],
        out_shape=[
            jax.ShapeDtypeStruct((co, nb * _LB), jnp.bfloat16),
            jax.ShapeDtypeStruct((co, 1), jnp.float32),
            jax.ShapeDtypeStruct((co, 1), jnp.float32),
        ],
    )(sp, st, tex, w_sp, w_st, w_tx)


def _norm_relu_cn(y, a, c, n):
    nb = -(-n // _LB)
    co = y.shape[0]

    def body(y_ref, a_ref, c_ref, o_ref):
        y = y_ref[...].astype(jnp.float32)
        o_ref[...] = jnp.maximum(a_ref[...] * y + c_ref[...], 0.0)

    return pl.pallas_call(
        body,
        grid=(nb,),
        in_specs=[
            pl.BlockSpec((co, _LB), lambda i: (0, i)),
            pl.BlockSpec((co, 1), lambda i: (0, 0)),
            pl.BlockSpec((co, 1), lambda i: (0, 0)),
        ],
        out_specs=pl.BlockSpec((co, _LB), lambda i: (0, i)),
        out_shape=jax.ShapeDtypeStruct((co, n), jnp.float32),
    )(y, a, c)


def _neighbor_conv_stats(st1d, gath1d, wbig, n2, npad):
    nb = -(-n2 // _RBP)
    kstride = npad * 64 // (_RBP * 128)
    blk = _RBP * 128

    def body(f_ref, g0_ref, g1_ref, g2_ref, w_ref, y_ref, s1_ref, s2_ref):
        i = pl.program_id(0)
        f = jnp.reshape(f_ref[...], (_RBP, 128))
        n0 = jnp.reshape(g0_ref[...], (_RBP, 128))
        n1 = jnp.reshape(g1_ref[...], (_RBP, 128))
        n2_ = jnp.reshape(g2_ref[...], (_RBP, 128))
        s_sum = n0 + n1 + n2_
        s_dif = jnp.abs(n2_ - n1) + 2.0 * jnp.abs(n1 - n0)
        s_div = jnp.abs(n0 - f) + jnp.abs(n1 - f) + jnp.abs(n2_ - f)
        z = jnp.concatenate([f, s_sum, s_dif, s_div], axis=1)
        y = lax.dot_general(z, w_ref[...], (((1,), (0,)), ((), ())),
                            preferred_element_type=jnp.float32,
                            precision=_PREC)
        y_ref[...] = y.astype(jnp.bfloat16)
        ids = i * _RBP + lax.broadcasted_iota(jnp.int32, y.shape, 0)
        ym = jnp.where(ids < n2, y, 0.0)

        @pl.when(i == 0)
        def _():
            s1_ref[...] = jnp.zeros_like(s1_ref)
            s2_ref[...] = jnp.zeros_like(s2_ref)

        s1_ref[...] += jnp.sum(ym, axis=0, keepdims=True)
        s2_ref[...] += jnp.sum(ym * ym, axis=0, keepdims=True)

    return pl.pallas_call(
        body,
        grid=(nb,),
        in_specs=[
            pl.BlockSpec((blk,), lambda i: (i,)),
            pl.BlockSpec((blk,), lambda i: (i,)),
            pl.BlockSpec((blk,), lambda i: (i + kstride,)),
            pl.BlockSpec((blk,), lambda i: (i + 2 * kstride,)),
            pl.BlockSpec((512, 128), lambda i: (0, 0)),
        ],
        out_specs=[
            pl.BlockSpec((_RBP, 128), lambda i: (i, 0)),
            pl.BlockSpec((1, 128), lambda i: (0, 0)),
            pl.BlockSpec((1, 128), lambda i: (0, 0)),
        ],
        out_shape=[
            jax.ShapeDtypeStruct((nb * _RBP, 128), jnp.bfloat16),
            jax.ShapeDtypeStruct((1, 128), jnp.float32),
            jax.ShapeDtypeStruct((1, 128), jnp.float32),
        ],
    )(st1d, gath1d, gath1d, gath1d, wbig)


def _norm_relu_conv_stats(y2, a, c, wbd, n2):
    nb = -(-n2 // _RBP)

    def body(y_ref, a_ref, c_ref, w_ref, y3_ref, s1_ref, s2_ref):
        i = pl.program_id(0)
        y2f = y_ref[...].astype(jnp.float32)
        st1 = jnp.maximum(a_ref[...] * y2f + c_ref[...], 0.0)
        y3 = lax.dot_general(st1, w_ref[...], (((1,), (0,)), ((), ())),
                             preferred_element_type=jnp.float32,
                             precision=_PREC)
        y3_ref[...] = y3.astype(jnp.bfloat16)
        ids = i * _RBP + lax.broadcasted_iota(jnp.int32, y3.shape, 0)
        ym = jnp.where(ids < n2, y3, 0.0)

        @pl.when(i == 0)
        def _():
            s1_ref[...] = jnp.zeros_like(s1_ref)
            s2_ref[...] = jnp.zeros_like(s2_ref)

        s1_ref[...] += jnp.sum(ym, axis=0, keepdims=True)
        s2_ref[...] += jnp.sum(ym * ym, axis=0, keepdims=True)

    return pl.pallas_call(
        body,
        grid=(nb,),
        in_specs=[
            pl.BlockSpec((_RBP, 128), lambda i: (i, 0)),
            pl.BlockSpec((1, 128), lambda i: (0, 0)),
            pl.BlockSpec((1, 128), lambda i: (0, 0)),
            pl.BlockSpec((128, 128), lambda i: (0, 0)),
        ],
        out_specs=[
            pl.BlockSpec((_RBP, 128), lambda i: (i, 0)),
            pl.BlockSpec((1, 128), lambda i: (0, 0)),
            pl.BlockSpec((1, 128), lambda i: (0, 0)),
        ],
        out_shape=[
            jax.ShapeDtypeStruct((nb * _RBP, 128), jnp.bfloat16),
            jax.ShapeDtypeStruct((1, 128), jnp.float32),
            jax.ShapeDtypeStruct((1, 128), jnp.float32),
        ],
    )(y2, a, c, wbd)


def _norm_relu_packed(y, a, c, n2):
    nb = -(-n2 // _RBP)

    def body(y_ref, a_ref, c_ref, o_ref):
        y3 = y_ref[...].astype(jnp.float32)
        o_ref[...] = jnp.maximum(a_ref[...] * y3 + c_ref[...], 0.0)

    return pl.pallas_call(
        body,
        grid=(nb,),
        in_specs=[
            pl.BlockSpec((_RBP, 128), lambda i: (i, 0)),
            pl.BlockSpec((1, 128), lambda i: (0, 0)),
            pl.BlockSpec((1, 128), lambda i: (0, 0)),
        ],
        out_specs=pl.BlockSpec((_RBP, 128), lambda i: (i, 0)),
        out_shape=jax.ShapeDtypeStruct((n2, 128), jnp.float32),
    )(y, a, c)


def _bn_coefs(s1, s2, gamma, beta, n):
    m = s1 / n
    v = s2 / n - m * m
    a = gamma * lax.rsqrt(v + _EPS)
    return a, beta - a * m


def kernel(spatial_fea, structural_fea, tex_fea, neighbor_index,
           W1, b1, g1, be1, W2, b2, g2, be2, W3, b3, g3, be3):
    B, c_sp, N = spatial_fea.shape
    c_st = structural_fea.shape[1]
    del b1, b2, b3

    sp = spatial_fea[0]
    st = structural_fea[0]
    tex = tex_fea[0]

    unit = 32 * _GRP // 3 if (32 * _GRP) % 3 == 0 else 32 * _GRP
    npad = -(-N // unit) * unit
    st1d = st.T.reshape(-1)
    stT = st1d.reshape(N, 64)
    zpad = jnp.zeros((npad - N,), jnp.int32)
    nix = neighbor_index[0]
    idx_flat = jnp.concatenate(
        [nix[:, 0], zpad, nix[:, 1], zpad, nix[:, 2], zpad])
    gath = _gather_rows(stT, idx_flat)
    gath1d = gath.reshape(-1)

    y1, s1, q1 = _conv_stats_cn(sp, st, tex, W1[:, :c_sp],
                                W1[:, c_sp:c_sp + c_st],
                                W1[:, c_sp + c_st:], N)
    a1, c1 = _bn_coefs(s1[:, 0], q1[:, 0], g1, be1, N)
    out_sp = _norm_relu_cn(y1, a1.reshape(-1, 1), c1.reshape(-1, 1), N)

    gath1d, out_sp = lax.optimization_barrier((gath1d, out_sp))

    n2 = N // 2
    eye2 = jnp.eye(2, dtype=jnp.float32)
    W2T = W2.T
    wbig = jnp.concatenate(
        [jnp.kron(eye2, W2T[k * c_st:(k + 1) * c_st]) for k in range(4)],
        axis=0)
    y2, s2, q2 = _neighbor_conv_stats(st1d, gath1d, wbig, n2, npad)
    a2, c2 = _bn_coefs(s2[0, :64] + s2[0, 64:], q2[0, :64] + q2[0, 64:],
                       g2, be2, N)
    a2p = jnp.tile(a2, 2).reshape(1, 128)
    c2p = jnp.tile(c2, 2).reshape(1, 128)
    w3bd = jnp.kron(eye2, W3.T)
    y3, s3, q3 = _norm_relu_conv_stats(y2, a2p, c2p, w3bd, n2)
    a3, c3 = _bn_coefs(s3[0, :64] + s3[0, 64:], q3[0, :64] + q3[0, 64:],
                       g3, be3, N)
    outP = _norm_relu_packed(y3, jnp.tile(a3, 2).reshape(1, 128),
                             jnp.tile(c3, 2).reshape(1, 128), n2)
    out_st = outP.reshape(N, 64).T

    return out_sp[None], out_st[None]


# speedup vs baseline: 1.0157x; 1.0157x over previous
import functools

import jax
import jax.numpy as jnp
from jax import lax
from jax.experimental import pallas as pl
from jax.experimental.pallas import tpu as pltpu
from jax.experimental.pallas import tpu_sc as plsc

_EPS = 1e-5
_LB = 4096
_RBP = 2048
_CH = 128
_CPG = 5
_GRP = _CH * _CPG
_NBUF = 3
_PREC = lax.Precision.DEFAULT


def _gather_rows(table, idx_flat):
    G = idx_flat.shape[0]
    info = plsc.get_sparse_core_info()
    NC, NS = info.num_cores, info.num_subcores
    NW = NC * NS
    per_w = G // NW
    n_groups = per_w // _GRP
    assert per_w % _GRP == 0 and G % NW == 0
    D = table.shape[1]
    mesh = plsc.VectorSubcoreMesh(core_axis_name="c", subcore_axis_name="s")

    @functools.partial(
        pl.kernel,
        mesh=mesh,
        compiler_params=pltpu.CompilerParams(use_tc_tiling_on_sc=False),
        cost_estimate=pl.CostEstimate(
            flops=0, bytes_accessed=int(G * D * 4 * 2), transcendentals=0),
        out_type=jax.ShapeDtypeStruct((G, D), jnp.float32),
        scratch_types=[
            pltpu.VMEM((_GRP,), jnp.int32),
            pltpu.VMEM((_GRP,), jnp.int32),
            pltpu.VMEM((_GRP,), jnp.int32),
            pltpu.VMEM((_GRP, D), jnp.float32),
            pltpu.VMEM((_GRP, D), jnp.float32),
            pltpu.VMEM((_GRP, D), jnp.float32),
            pltpu.SemaphoreType.DMA,
            pltpu.SemaphoreType.DMA,
            pltpu.SemaphoreType.DMA,
            pltpu.SemaphoreType.DMA,
            pltpu.SemaphoreType.DMA,
            pltpu.SemaphoreType.DMA,
        ],
    )
    def k(table_hbm, idx_hbm, out_hbm, i0, i1, i2, r0, r1, r2,
          gs0, gs1, gs2, ss0, ss1, ss2):
        idxs = [i0, i1, i2]
        rows = [r0, r1, r2]
        gsem = [gs0, gs1, gs2]
        ssem = [ss0, ss1, ss2]
        wid = lax.axis_index("s") * NC + lax.axis_index("c")
        base_w = wid * per_w
        gcopies = [None] * _NBUF
        stores = [None] * _NBUF
        for g in range(n_groups):
            b = g % _NBUF
            if g >= _NBUF:
                stores[b].wait()
            pltpu.sync_copy(idx_hbm.at[pl.ds(base_w + g * _GRP, _GRP)],
                            idxs[b])
            cs = []
            for j in range(_CPG):
                cs.append(pltpu.async_copy(
                    table_hbm.at[idxs[b].at[pl.ds(j * _CH, _CH)]],
                    rows[b].at[pl.ds(j * _CH, _CH)], gsem[b]))
            gcopies[b] = cs
            if g >= 1:
                pb = (g - 1) % _NBUF
                for c in gcopies[pb]:
                    c.wait()
                stores[pb] = pltpu.async_copy(
                    rows[pb],
                    out_hbm.at[pl.ds(base_w + (g - 1) * _GRP, _GRP)],
                    ssem[pb])
        lb = (n_groups - 1) % _NBUF
        for c in gcopies[lb]:
            c.wait()
        stores[lb] = pltpu.async_copy(
            rows[lb],
            out_hbm.at[pl.ds(base_w + (n_groups - 1) * _GRP, _GRP)],
            ssem[lb])
        for b in range(_NBUF):
            stores[b].wait()

    return k(table, idx_flat)


def _conv_stats_cn(sp, st, tex, w_sp, w_st, w_tx, n):
    nb = -(-n // _LB)
    co = w_sp.shape[0]

    def body(sp_ref, st_ref, tex_ref, wa_ref, wb_ref, wc_ref,
             y_ref, s1_ref, s2_ref):
        i = pl.program_id(0)
        dn = (((1,), (0,)), ((), ()))
        y = lax.dot_general(wa_ref[...], sp_ref[...], dn,
                            preferred_element_type=jnp.float32,
                            precision=_PREC)
        y += lax.dot_general(wb_ref[...], st_ref[...], dn,
                             preferred_element_type=jnp.float32,
                             precision=_PREC)
        y += lax.dot_general(wc_ref[...], tex_ref[...], dn,
                             preferred_element_type=jnp.float32,
                             precision=_PREC)
        y_ref[...] = y.astype(jnp.bfloat16)
        ids = i * _LB + lax.broadcasted_iota(jnp.int32, y.shape, 1)
        ym = jnp.where(ids < n, y, 0.0)

        @pl.when(i == 0)
        def _():
            s1_ref[...] = jnp.zeros_like(s1_ref)
            s2_ref[...] = jnp.zeros_like(s2_ref)

        s1_ref[...] += jnp.sum(ym, axis=1, keepdims=True)
        s2_ref[...] += jnp.sum(ym * ym, axis=1, keepdims=True)

    c_sp, c_st, c_tx = sp.shape[0], st.shape[0], tex.shape[0]
    return pl.pallas_call(
        body,
        grid=(nb,),
        in_specs=[
            pl.BlockSpec((c_sp, _LB), lambda i: (0, i)),
            pl.BlockSpec((c_st, _LB), lambda i: (0, i)),
            pl.BlockSpec((c_tx, _LB), lambda i: (0, i)),
            pl.BlockSpec((co, c_sp), lambda i: (0, 0)),
            pl.BlockSpec((co, c_st), lambda i: (0, 0)),
            pl.BlockSpec((co, c_tx), lambda i: (0, 0)),
        ],
        out_specs=[
            pl.BlockSpec((co, _LB), lambda i: (0, i)),
            pl.BlockSpec((co, 1), lambda i: (0, 0)),
            pl.BlockSpec((co, 1), lambda i: (0, 0)),
        ],
        out_shape=[
            jax.ShapeDtypeStruct((co, nb * _LB), jnp.bfloat16),
            jax.ShapeDtypeStruct((co, 1), jnp.float32),
            jax.ShapeDtypeStruct((co, 1), jnp.float32),
        ],
    )(sp, st, tex, w_sp, w_st, w_tx)


def _norm_relu_cn(y, a, c, n):
    nb = -(-n // _LB)
    co = y.shape[0]

    def body(y_ref, a_ref, c_ref, o_ref):
        y = y_ref[...].astype(jnp.float32)
        o_ref[...] = jnp.maximum(a_ref[...] * y + c_ref[...], 0.0)

    return pl.pallas_call(
        body,
        grid=(nb,),
        in_specs=[
            pl.BlockSpec((co, _LB), lambda i: (0, i)),
            pl.BlockSpec((co, 1), lambda i: (0, 0)),
            pl.BlockSpec((co, 1), lambda i: (0, 0)),
        ],
        out_specs=pl.BlockSpec((co, _LB), lambda i: (0, i)),
        out_shape=jax.ShapeDtypeStruct((co, n), jnp.float32),
    )(y, a, c)


def _neighbor_conv_stats(st1d, gath1d, wbig, n2, npad):
    nb = -(-n2 // _RBP)
    kstride = npad * 64 // (_RBP * 128)
    blk = _RBP * 128

    def body(f_ref, g0_ref, g1_ref, g2_ref, w_ref, y_ref, s1_ref, s2_ref):
        i = pl.program_id(0)
        f = jnp.reshape(f_ref[...], (_RBP, 128))
        n0 = jnp.reshape(g0_ref[...], (_RBP, 128))
        n1 = jnp.reshape(g1_ref[...], (_RBP, 128))
        n2_ = jnp.reshape(g2_ref[...], (_RBP, 128))
        s_sum = n0 + n1 + n2_
        s_dif = jnp.abs(n2_ - n1) + 2.0 * jnp.abs(n1 - n0)
        s_div = jnp.abs(n0 - f) + jnp.abs(n1 - f) + jnp.abs(n2_ - f)
        z = jnp.concatenate([f, s_sum, s_dif, s_div], axis=1)
        y = lax.dot_general(z, w_ref[...], (((1,), (0,)), ((), ())),
                            preferred_element_type=jnp.float32,
                            precision=_PREC)
        y_ref[...] = y.astype(jnp.bfloat16)
        ids = i * _RBP + lax.broadcasted_iota(jnp.int32, y.shape, 0)
        ym = jnp.where(ids < n2, y, 0.0)

        @pl.when(i == 0)
        def _():
            s1_ref[...] = jnp.zeros_like(s1_ref)
            s2_ref[...] = jnp.zeros_like(s2_ref)

        s1_ref[...] += jnp.sum(ym, axis=0, keepdims=True)
        s2_ref[...] += jnp.sum(ym * ym, axis=0, keepdims=True)

    return pl.pallas_call(
        body,
        grid=(nb,),
        in_specs=[
            pl.BlockSpec((blk,), lambda i: (i,)),
            pl.BlockSpec((blk,), lambda i: (i,)),
            pl.BlockSpec((blk,), lambda i: (i + kstride,)),
            pl.BlockSpec((blk,), lambda i: (i + 2 * kstride,)),
            pl.BlockSpec((512, 128), lambda i: (0, 0)),
        ],
        out_specs=[
            pl.BlockSpec((_RBP, 128), lambda i: (i, 0)),
            pl.BlockSpec((1, 128), lambda i: (0, 0)),
            pl.BlockSpec((1, 128), lambda i: (0, 0)),
        ],
        out_shape=[
            jax.ShapeDtypeStruct((nb * _RBP, 128), jnp.bfloat16),
            jax.ShapeDtypeStruct((1, 128), jnp.float32),
            jax.ShapeDtypeStruct((1, 128), jnp.float32),
        ],
    )(st1d, gath1d, gath1d, gath1d, wbig)


def _norm_relu_conv_stats(y2, a, c, wbd, n2):
    nb = -(-n2 // _RBP)

    def body(y_ref, a_ref, c_ref, w_ref, y3_ref, s1_ref, s2_ref):
        i = pl.program_id(0)
        y2f = y_ref[...].astype(jnp.float32)
        st1 = jnp.maximum(a_ref[...] * y2f + c_ref[...], 0.0)
        y3 = lax.dot_general(st1, w_ref[...], (((1,), (0,)), ((), ())),
                             preferred_element_type=jnp.float32,
                             precision=_PREC)
        y3_ref[...] = y3.astype(jnp.bfloat16)
        ids = i * _RBP + lax.broadcasted_iota(jnp.int32, y3.shape, 0)
        ym = jnp.where(ids < n2, y3, 0.0)

        @pl.when(i == 0)
        def _():
            s1_ref[...] = jnp.zeros_like(s1_ref)
            s2_ref[...] = jnp.zeros_like(s2_ref)

        s1_ref[...] += jnp.sum(ym, axis=0, keepdims=True)
        s2_ref[...] += jnp.sum(ym * ym, axis=0, keepdims=True)

    return pl.pallas_call(
        body,
        grid=(nb,),
        in_specs=[
            pl.BlockSpec((_RBP, 128), lambda i: (i, 0)),
            pl.BlockSpec((1, 128), lambda i: (0, 0)),
            pl.BlockSpec((1, 128), lambda i: (0, 0)),
            pl.BlockSpec((128, 128), lambda i: (0, 0)),
        ],
        out_specs=[
            pl.BlockSpec((_RBP, 128), lambda i: (i, 0)),
            pl.BlockSpec((1, 128), lambda i: (0, 0)),
            pl.BlockSpec((1, 128), lambda i: (0, 0)),
        ],
        out_shape=[
            jax.ShapeDtypeStruct((nb * _RBP, 128), jnp.bfloat16),
            jax.ShapeDtypeStruct((1, 128), jnp.float32),
            jax.ShapeDtypeStruct((1, 128), jnp.float32),
        ],
    )(y2, a, c, wbd)


def _norm_relu_packed(y, a, c, n2):
    nb = -(-n2 // _RBP)

    def body(y_ref, a_ref, c_ref, o_ref):
        y3 = y_ref[...].astype(jnp.float32)
        o_ref[...] = jnp.maximum(a_ref[...] * y3 + c_ref[...], 0.0)

    return pl.pallas_call(
        body,
        grid=(nb,),
        in_specs=[
            pl.BlockSpec((_RBP, 128), lambda i: (i, 0)),
            pl.BlockSpec((1, 128), lambda i: (0, 0)),
            pl.BlockSpec((1, 128), lambda i: (0, 0)),
        ],
        out_specs=pl.BlockSpec((_RBP, 128), lambda i: (i, 0)),
        out_shape=jax.ShapeDtypeStruct((n2, 128), jnp.float32),
    )(y, a, c)


def _bn_coefs(s1, s2, gamma, beta, n):
    m = s1 / n
    v = s2 / n - m * m
    a = gamma * lax.rsqrt(v + _EPS)
    return a, beta - a * m


def kernel(spatial_fea, structural_fea, tex_fea, neighbor_index,
           W1, b1, g1, be1, W2, b2, g2, be2, W3, b3, g3, be3):
    B, c_sp, N = spatial_fea.shape
    c_st = structural_fea.shape[1]
    del b1, b2, b3

    sp = spatial_fea[0]
    st = structural_fea[0]
    tex = tex_fea[0]

    unit = 32 * _GRP // 3 if (32 * _GRP) % 3 == 0 else 32 * _GRP
    npad = -(-N // unit) * unit
    st1d = st.T.reshape(-1)
    stT = st1d.reshape(N, 64)
    zpad = jnp.zeros((npad - N,), jnp.int32)
    nix = neighbor_index[0]
    idx_flat = jnp.concatenate(
        [nix[:, 0], zpad, nix[:, 1], zpad, nix[:, 2], zpad])
    gath = _gather_rows(stT, idx_flat)
    gath1d = gath.reshape(-1)

    y1, s1, q1 = _conv_stats_cn(sp, st, tex, W1[:, :c_sp],
                                W1[:, c_sp:c_sp + c_st],
                                W1[:, c_sp + c_st:], N)
    a1, c1 = _bn_coefs(s1[:, 0], q1[:, 0], g1, be1, N)
    out_sp = _norm_relu_cn(y1, a1.reshape(-1, 1), c1.reshape(-1, 1), N)

    gath1d, out_sp = lax.optimization_barrier((gath1d, out_sp))

    n2 = N // 2
    eye2 = jnp.eye(2, dtype=jnp.float32)
    W2T = W2.T
    wbig = jnp.concatenate(
        [jnp.kron(eye2, W2T[k * c_st:(k + 1) * c_st]) for k in range(4)],
        axis=0)
    y2, s2, q2 = _neighbor_conv_stats(st1d, gath1d, wbig, n2, npad)
    a2, c2 = _bn_coefs(s2[0, :64] + s2[0, 64:], q2[0, :64] + q2[0, 64:],
                       g2, be2, N)
    a2p = jnp.tile(a2, 2).reshape(1, 128)
    c2p = jnp.tile(c2, 2).reshape(1, 128)
    w3bd = jnp.kron(eye2, W3.T)
    y3, s3, q3 = _norm_relu_conv_stats(y2, a2p, c2p, w3bd, n2)
    a3, c3 = _bn_coefs(s3[0, :64] + s3[0, 64:], q3[0, :64] + q3[0, 64:],
                       g3, be3, N)
    outP = _norm_relu_packed(y3, jnp.tile(a3, 2).reshape(1, 128),
                             jnp.tile(c3, 2).reshape(1, 128), n2)
    out_st = outP.reshape(N, 64).T

    return out_sp[None], out_st[None]


# speedup vs baseline: 1.0375x; 1.0214x over previous
import functools

import jax
import jax.numpy as jnp
from jax import lax
from jax.experimental import pallas as pl
from jax.experimental.pallas import tpu as pltpu
from jax.experimental.pallas import tpu_sc as plsc

_EPS = 1e-5
_LB = 4096
_RBP = 2048
_RBP2 = 4096
_CH = 128
_CPG = 5
_GRP = _CH * _CPG
_NBUF = 3
_PREC = lax.Precision.DEFAULT


def _gather_rows(table, idx_flat):
    G = idx_flat.shape[0]
    info = plsc.get_sparse_core_info()
    NC, NS = info.num_cores, info.num_subcores
    NW = NC * NS
    per_w = G // NW
    n_groups = per_w // _GRP
    assert per_w % _GRP == 0 and G % NW == 0
    D = table.shape[1]
    mesh = plsc.VectorSubcoreMesh(core_axis_name="c", subcore_axis_name="s")

    @functools.partial(
        pl.kernel,
        mesh=mesh,
        compiler_params=pltpu.CompilerParams(use_tc_tiling_on_sc=False),
        cost_estimate=pl.CostEstimate(
            flops=0, bytes_accessed=int(G * D * 4 * 2), transcendentals=0),
        out_type=jax.ShapeDtypeStruct((G, D), jnp.float32),
        scratch_types=[
            pltpu.VMEM((_GRP,), jnp.int32),
            pltpu.VMEM((_GRP,), jnp.int32),
            pltpu.VMEM((_GRP,), jnp.int32),
            pltpu.VMEM((_GRP, D), jnp.float32),
            pltpu.VMEM((_GRP, D), jnp.float32),
            pltpu.VMEM((_GRP, D), jnp.float32),
            pltpu.SemaphoreType.DMA,
            pltpu.SemaphoreType.DMA,
            pltpu.SemaphoreType.DMA,
            pltpu.SemaphoreType.DMA,
            pltpu.SemaphoreType.DMA,
            pltpu.SemaphoreType.DMA,
        ],
    )
    def k(table_hbm, idx_hbm, out_hbm, i0, i1, i2, r0, r1, r2,
          gs0, gs1, gs2, ss0, ss1, ss2):
        idxs = [i0, i1, i2]
        rows = [r0, r1, r2]
        gsem = [gs0, gs1, gs2]
        ssem = [ss0, ss1, ss2]
        wid = lax.axis_index("s") * NC + lax.axis_index("c")
        base_w = wid * per_w
        gcopies = [None] * _NBUF
        stores = [None] * _NBUF
        for g in range(n_groups):
            b = g % _NBUF
            if g >= _NBUF:
                stores[b].wait()
            pltpu.sync_copy(idx_hbm.at[pl.ds(base_w + g * _GRP, _GRP)],
                            idxs[b])
            cs = []
            for j in range(_CPG):
                cs.append(pltpu.async_copy(
                    table_hbm.at[idxs[b].at[pl.ds(j * _CH, _CH)]],
                    rows[b].at[pl.ds(j * _CH, _CH)], gsem[b]))
            gcopies[b] = cs
            if g >= 1:
                pb = (g - 1) % _NBUF
                for c in gcopies[pb]:
                    c.wait()
                stores[pb] = pltpu.async_copy(
                    rows[pb],
                    out_hbm.at[pl.ds(base_w + (g - 1) * _GRP, _GRP)],
                    ssem[pb])
        lb = (n_groups - 1) % _NBUF
        for c in gcopies[lb]:
            c.wait()
        stores[lb] = pltpu.async_copy(
            rows[lb],
            out_hbm.at[pl.ds(base_w + (n_groups - 1) * _GRP, _GRP)],
            ssem[lb])
        for b in range(_NBUF):
            stores[b].wait()

    return k(table, idx_flat)


def _conv_stats_cn(sp, st, tex, w_sp, w_st, w_tx, n):
    nb = -(-n // _LB)
    co = w_sp.shape[0]

    def body(sp_ref, st_ref, tex_ref, wa_ref, wb_ref, wc_ref,
             y_ref, s1_ref, s2_ref):
        i = pl.program_id(0)
        dn = (((1,), (0,)), ((), ()))
        y = lax.dot_general(wa_ref[...], sp_ref[...], dn,
                            preferred_element_type=jnp.float32,
                            precision=_PREC)
        y += lax.dot_general(wb_ref[...], st_ref[...], dn,
                             preferred_element_type=jnp.float32,
                             precision=_PREC)
        y += lax.dot_general(wc_ref[...], tex_ref[...], dn,
                             preferred_element_type=jnp.float32,
                             precision=_PREC)
        y_ref[...] = y.astype(jnp.bfloat16)
        ids = i * _LB + lax.broadcasted_iota(jnp.int32, y.shape, 1)
        ym = jnp.where(ids < n, y, 0.0)

        @pl.when(i == 0)
        def _():
            s1_ref[...] = jnp.zeros_like(s1_ref)
            s2_ref[...] = jnp.zeros_like(s2_ref)

        s1_ref[...] += jnp.sum(ym, axis=1, keepdims=True)
        s2_ref[...] += jnp.sum(ym * ym, axis=1, keepdims=True)

    c_sp, c_st, c_tx = sp.shape[0], st.shape[0], tex.shape[0]
    return pl.pallas_call(
        body,
        grid=(nb,),
        in_specs=[
            pl.BlockSpec((c_sp, _LB), lambda i: (0, i)),
            pl.BlockSpec((c_st, _LB), lambda i: (0, i)),
            pl.BlockSpec((c_tx, _LB), lambda i: (0, i)),
            pl.BlockSpec((co, c_sp), lambda i: (0, 0)),
            pl.BlockSpec((co, c_st), lambda i: (0, 0)),
            pl.BlockSpec((co, c_tx), lambda i: (0, 0)),
        ],
        out_specs=[
            pl.BlockSpec((co, _LB), lambda i: (0, i)),
            pl.BlockSpec((co, 1), lambda i: (0, 0)),
            pl.BlockSpec((co, 1), lambda i: (0, 0)),
        ],
        out_shape=[
            jax.ShapeDtypeStruct((co, nb * _LB), jnp.bfloat16),
            jax.ShapeDtypeStruct((co, 1), jnp.float32),
            jax.ShapeDtypeStruct((co, 1), jnp.float32),
        ],
    )(sp, st, tex, w_sp, w_st, w_tx)


def _norm_relu_cn(y, a, c, n):
    nb = -(-n // _LB)
    co = y.shape[0]

    def body(y_ref, a_ref, c_ref, o_ref):
        y = y_ref[...].astype(jnp.float32)
        o_ref[...] = jnp.maximum(a_ref[...] * y + c_ref[...], 0.0)

    return pl.pallas_call(
        body,
        grid=(nb,),
        in_specs=[
            pl.BlockSpec((co, _LB), lambda i: (0, i)),
            pl.BlockSpec((co, 1), lambda i: (0, 0)),
            pl.BlockSpec((co, 1), lambda i: (0, 0)),
        ],
        out_specs=pl.BlockSpec((co, _LB), lambda i: (0, i)),
        out_shape=jax.ShapeDtypeStruct((co, n), jnp.float32),
    )(y, a, c)


def _neighbor_conv_stats(st1d, gath1d, wbig, n2, npad):
    nb = -(-n2 // _RBP)
    kstride = npad * 64 // (_RBP * 128)
    blk = _RBP * 128

    def body(f_ref, g0_ref, g1_ref, g2_ref, w_ref, y_ref, s1_ref, s2_ref):
        i = pl.program_id(0)
        f = jnp.reshape(f_ref[...], (_RBP, 128))
        n0 = jnp.reshape(g0_ref[...], (_RBP, 128))
        n1 = jnp.reshape(g1_ref[...], (_RBP, 128))
        n2_ = jnp.reshape(g2_ref[...], (_RBP, 128))
        s_sum = n0 + n1 + n2_
        s_dif = jnp.abs(n2_ - n1) + 2.0 * jnp.abs(n1 - n0)
        s_div = jnp.abs(n0 - f) + jnp.abs(n1 - f) + jnp.abs(n2_ - f)
        z = jnp.concatenate([f, s_sum, s_dif, s_div], axis=1)
        y = lax.dot_general(z, w_ref[...], (((1,), (0,)), ((), ())),
                            preferred_element_type=jnp.float32,
                            precision=_PREC)
        y_ref[...] = y.astype(jnp.bfloat16)
        ids = i * _RBP + lax.broadcasted_iota(jnp.int32, y.shape, 0)
        ym = jnp.where(ids < n2, y, 0.0)

        @pl.when(i == 0)
        def _():
            s1_ref[...] = jnp.zeros_like(s1_ref)
            s2_ref[...] = jnp.zeros_like(s2_ref)

        s1_ref[...] += jnp.sum(ym, axis=0, keepdims=True)
        s2_ref[...] += jnp.sum(ym * ym, axis=0, keepdims=True)

    return pl.pallas_call(
        body,
        grid=(nb,),
        in_specs=[
            pl.BlockSpec((blk,), lambda i: (i,)),
            pl.BlockSpec((blk,), lambda i: (i,)),
            pl.BlockSpec((blk,), lambda i: (i + kstride,)),
            pl.BlockSpec((blk,), lambda i: (i + 2 * kstride,)),
            pl.BlockSpec((512, 128), lambda i: (0, 0)),
        ],
        out_specs=[
            pl.BlockSpec((_RBP, 128), lambda i: (i, 0)),
            pl.BlockSpec((1, 128), lambda i: (0, 0)),
            pl.BlockSpec((1, 128), lambda i: (0, 0)),
        ],
        out_shape=[
            jax.ShapeDtypeStruct((nb * _RBP, 128), jnp.bfloat16),
            jax.ShapeDtypeStruct((1, 128), jnp.float32),
            jax.ShapeDtypeStruct((1, 128), jnp.float32),
        ],
    )(st1d, gath1d, gath1d, gath1d, wbig)


def _norm_relu_conv_stats(y2, a, c, wbd, n2):
    nb = -(-n2 // _RBP2)

    def body(y_ref, a_ref, c_ref, w_ref, y3_ref, s1_ref, s2_ref):
        i = pl.program_id(0)
        y2f = y_ref[...].astype(jnp.float32)
        st1 = jnp.maximum(a_ref[...] * y2f + c_ref[...], 0.0)
        y3 = lax.dot_general(st1, w_ref[...], (((1,), (0,)), ((), ())),
                             preferred_element_type=jnp.float32,
                             precision=_PREC)
        y3_ref[...] = y3.astype(jnp.bfloat16)
        ids = i * _RBP2 + lax.broadcasted_iota(jnp.int32, y3.shape, 0)
        ym = jnp.where(ids < n2, y3, 0.0)

        @pl.when(i == 0)
        def _():
            s1_ref[...] = jnp.zeros_like(s1_ref)
            s2_ref[...] = jnp.zeros_like(s2_ref)

        s1_ref[...] += jnp.sum(ym, axis=0, keepdims=True)
        s2_ref[...] += jnp.sum(ym * ym, axis=0, keepdims=True)

    return pl.pallas_call(
        body,
        grid=(nb,),
        in_specs=[
            pl.BlockSpec((_RBP2, 128), lambda i: (i, 0)),
            pl.BlockSpec((1, 128), lambda i: (0, 0)),
            pl.BlockSpec((1, 128), lambda i: (0, 0)),
            pl.BlockSpec((128, 128), lambda i: (0, 0)),
        ],
        out_specs=[
            pl.BlockSpec((_RBP2, 128), lambda i: (i, 0)),
            pl.BlockSpec((1, 128), lambda i: (0, 0)),
            pl.BlockSpec((1, 128), lambda i: (0, 0)),
        ],
        out_shape=[
            jax.ShapeDtypeStruct((nb * _RBP2, 128), jnp.bfloat16),
            jax.ShapeDtypeStruct((1, 128), jnp.float32),
            jax.ShapeDtypeStruct((1, 128), jnp.float32),
        ],
    )(y2, a, c, wbd)


def _norm_relu_packed(y, a, c, n2):
    nb = -(-n2 // _RBP2)

    def body(y_ref, a_ref, c_ref, o_ref):
        y3 = y_ref[...].astype(jnp.float32)
        o_ref[...] = jnp.maximum(a_ref[...] * y3 + c_ref[...], 0.0)

    return pl.pallas_call(
        body,
        grid=(nb,),
        in_specs=[
            pl.BlockSpec((_RBP2, 128), lambda i: (i, 0)),
            pl.BlockSpec((1, 128), lambda i: (0, 0)),
            pl.BlockSpec((1, 128), lambda i: (0, 0)),
        ],
        out_specs=pl.BlockSpec((_RBP2, 128), lambda i: (i, 0)),
        out_shape=jax.ShapeDtypeStruct((n2, 128), jnp.float32),
    )(y, a, c)


def _bn_coefs(s1, s2, gamma, beta, n):
    m = s1 / n
    v = s2 / n - m * m
    a = gamma * lax.rsqrt(v + _EPS)
    return a, beta - a * m


def kernel(spatial_fea, structural_fea, tex_fea, neighbor_index,
           W1, b1, g1, be1, W2, b2, g2, be2, W3, b3, g3, be3):
    B, c_sp, N = spatial_fea.shape
    c_st = structural_fea.shape[1]
    del b1, b2, b3

    sp = spatial_fea[0]
    st = structural_fea[0]
    tex = tex_fea[0]

    unit = 32 * _GRP // 3 if (32 * _GRP) % 3 == 0 else 32 * _GRP
    npad = -(-N // unit) * unit
    st1d = st.T.reshape(-1)
    stT = st1d.reshape(N, 64)
    zpad = jnp.zeros((npad - N,), jnp.int32)
    nix = neighbor_index[0]
    idx_flat = jnp.concatenate(
        [nix[:, 0], zpad, nix[:, 1], zpad, nix[:, 2], zpad])
    gath = _gather_rows(stT, idx_flat)
    gath1d = gath.reshape(-1)

    y1, s1, q1 = _conv_stats_cn(sp, st, tex, W1[:, :c_sp],
                                W1[:, c_sp:c_sp + c_st],
                                W1[:, c_sp + c_st:], N)
    a1, c1 = _bn_coefs(s1[:, 0], q1[:, 0], g1, be1, N)
    out_sp = _norm_relu_cn(y1, a1.reshape(-1, 1), c1.reshape(-1, 1), N)

    gath1d, out_sp = lax.optimization_barrier((gath1d, out_sp))

    n2 = N // 2
    eye2 = jnp.eye(2, dtype=jnp.float32)
    W2T = W2.T
    wbig = jnp.concatenate(
        [jnp.kron(eye2, W2T[k * c_st:(k + 1) * c_st]) for k in range(4)],
        axis=0)
    y2, s2, q2 = _neighbor_conv_stats(st1d, gath1d, wbig, n2, npad)
    a2, c2 = _bn_coefs(s2[0, :64] + s2[0, 64:], q2[0, :64] + q2[0, 64:],
                       g2, be2, N)
    a2p = jnp.tile(a2, 2).reshape(1, 128)
    c2p = jnp.tile(c2, 2).reshape(1, 128)
    w3bd = jnp.kron(eye2, W3.T)
    y3, s3, q3 = _norm_relu_conv_stats(y2, a2p, c2p, w3bd, n2)
    a3, c3 = _bn_coefs(s3[0, :64] + s3[0, 64:], q3[0, :64] + q3[0, 64:],
                       g3, be3, N)
    outP = _norm_relu_packed(y3, jnp.tile(a3, 2).reshape(1, 128),
                             jnp.tile(c3, 2).reshape(1, 128), n2)
    out_st = outP.reshape(N, 64).T

    return out_sp[None], out_st[None]


# speedup vs baseline: 1.0474x; 1.0095x over previous
import functools

import jax
import jax.numpy as jnp
from jax import lax
from jax.experimental import pallas as pl
from jax.experimental.pallas import tpu as pltpu
from jax.experimental.pallas import tpu_sc as plsc

_EPS = 1e-5
_LB = 4096
_RBP = 2560
_RBP2 = 4096
_CH = 128
_CPG = 5
_GRP = _CH * _CPG
_NBUF = 3
_PREC = lax.Precision.DEFAULT


def _gather_rows(table, idx_flat):
    G = idx_flat.shape[0]
    info = plsc.get_sparse_core_info()
    NC, NS = info.num_cores, info.num_subcores
    NW = NC * NS
    per_w = G // NW
    n_groups = per_w // _GRP
    assert per_w % _GRP == 0 and G % NW == 0
    D = table.shape[1]
    mesh = plsc.VectorSubcoreMesh(core_axis_name="c", subcore_axis_name="s")

    @functools.partial(
        pl.kernel,
        mesh=mesh,
        compiler_params=pltpu.CompilerParams(use_tc_tiling_on_sc=False),
        cost_estimate=pl.CostEstimate(
            flops=0, bytes_accessed=int(G * D * 4 * 2), transcendentals=0),
        out_type=jax.ShapeDtypeStruct((G, D), jnp.float32),
        scratch_types=[
            pltpu.VMEM((_GRP,), jnp.int32),
            pltpu.VMEM((_GRP,), jnp.int32),
            pltpu.VMEM((_GRP,), jnp.int32),
            pltpu.VMEM((_GRP, D), jnp.float32),
            pltpu.VMEM((_GRP, D), jnp.float32),
            pltpu.VMEM((_GRP, D), jnp.float32),
            pltpu.SemaphoreType.DMA,
            pltpu.SemaphoreType.DMA,
            pltpu.SemaphoreType.DMA,
            pltpu.SemaphoreType.DMA,
            pltpu.SemaphoreType.DMA,
            pltpu.SemaphoreType.DMA,
        ],
    )
    def k(table_hbm, idx_hbm, out_hbm, i0, i1, i2, r0, r1, r2,
          gs0, gs1, gs2, ss0, ss1, ss2):
        idxs = [i0, i1, i2]
        rows = [r0, r1, r2]
        gsem = [gs0, gs1, gs2]
        ssem = [ss0, ss1, ss2]
        wid = lax.axis_index("s") * NC + lax.axis_index("c")
        base_w = wid * per_w
        gcopies = [None] * _NBUF
        stores = [None] * _NBUF
        for g in range(n_groups):
            b = g % _NBUF
            if g >= _NBUF:
                stores[b].wait()
            pltpu.sync_copy(idx_hbm.at[pl.ds(base_w + g * _GRP, _GRP)],
                            idxs[b])
            cs = []
            for j in range(_CPG):
                cs.append(pltpu.async_copy(
                    table_hbm.at[idxs[b].at[pl.ds(j * _CH, _CH)]],
                    rows[b].at[pl.ds(j * _CH, _CH)], gsem[b]))
            gcopies[b] = cs
            if g >= 1:
                pb = (g - 1) % _NBUF
                for c in gcopies[pb]:
                    c.wait()
                stores[pb] = pltpu.async_copy(
                    rows[pb],
                    out_hbm.at[pl.ds(base_w + (g - 1) * _GRP, _GRP)],
                    ssem[pb])
        lb = (n_groups - 1) % _NBUF
        for c in gcopies[lb]:
            c.wait()
        stores[lb] = pltpu.async_copy(
            rows[lb],
            out_hbm.at[pl.ds(base_w + (n_groups - 1) * _GRP, _GRP)],
            ssem[lb])
        for b in range(_NBUF):
            stores[b].wait()

    return k(table, idx_flat)


def _conv_stats_cn(sp, st, tex, w_sp, w_st, w_tx, n):
    nb = -(-n // _LB)
    co = w_sp.shape[0]

    def body(sp_ref, st_ref, tex_ref, wa_ref, wb_ref, wc_ref,
             y_ref, s1_ref, s2_ref):
        i = pl.program_id(0)
        dn = (((1,), (0,)), ((), ()))
        y = lax.dot_general(wa_ref[...], sp_ref[...], dn,
                            preferred_element_type=jnp.float32,
                            precision=_PREC)
        y += lax.dot_general(wb_ref[...], st_ref[...], dn,
                             preferred_element_type=jnp.float32,
                             precision=_PREC)
        y += lax.dot_general(wc_ref[...], tex_ref[...], dn,
                             preferred_element_type=jnp.float32,
                             precision=_PREC)
        y_ref[...] = y.astype(jnp.bfloat16)
        ids = i * _LB + lax.broadcasted_iota(jnp.int32, y.shape, 1)
        ym = jnp.where(ids < n, y, 0.0)

        @pl.when(i == 0)
        def _():
            s1_ref[...] = jnp.zeros_like(s1_ref)
            s2_ref[...] = jnp.zeros_like(s2_ref)

        s1_ref[...] += jnp.sum(ym, axis=1, keepdims=True)
        s2_ref[...] += jnp.sum(ym * ym, axis=1, keepdims=True)

    c_sp, c_st, c_tx = sp.shape[0], st.shape[0], tex.shape[0]
    return pl.pallas_call(
        body,
        grid=(nb,),
        in_specs=[
            pl.BlockSpec((c_sp, _LB), lambda i: (0, i)),
            pl.BlockSpec((c_st, _LB), lambda i: (0, i)),
            pl.BlockSpec((c_tx, _LB), lambda i: (0, i)),
            pl.BlockSpec((co, c_sp), lambda i: (0, 0)),
            pl.BlockSpec((co, c_st), lambda i: (0, 0)),
            pl.BlockSpec((co, c_tx), lambda i: (0, 0)),
        ],
        out_specs=[
            pl.BlockSpec((co, _LB), lambda i: (0, i)),
            pl.BlockSpec((co, 1), lambda i: (0, 0)),
            pl.BlockSpec((co, 1), lambda i: (0, 0)),
        ],
        out_shape=[
            jax.ShapeDtypeStruct((co, nb * _LB), jnp.bfloat16),
            jax.ShapeDtypeStruct((co, 1), jnp.float32),
            jax.ShapeDtypeStruct((co, 1), jnp.float32),
        ],
    )(sp, st, tex, w_sp, w_st, w_tx)


def _norm_relu_cn(y, a, c, n):
    nb = -(-n // _LB)
    co = y.shape[0]

    def body(y_ref, a_ref, c_ref, o_ref):
        y = y_ref[...].astype(jnp.float32)
        o_ref[...] = jnp.maximum(a_ref[...] * y + c_ref[...], 0.0)

    return pl.pallas_call(
        body,
        grid=(nb,),
        in_specs=[
            pl.BlockSpec((co, _LB), lambda i: (0, i)),
            pl.BlockSpec((co, 1), lambda i: (0, 0)),
            pl.BlockSpec((co, 1), lambda i: (0, 0)),
        ],
        out_specs=pl.BlockSpec((co, _LB), lambda i: (0, i)),
        out_shape=jax.ShapeDtypeStruct((co, n), jnp.float32),
    )(y, a, c)


def _neighbor_conv_stats(st1d, gath1d, wbig, n2, npad):
    nb = -(-n2 // _RBP)
    kstride = npad * 64 // (_RBP * 128)
    blk = _RBP * 128

    def body(f_ref, g0_ref, g1_ref, g2_ref, w_ref, y_ref, s1_ref, s2_ref):
        i = pl.program_id(0)
        f = jnp.reshape(f_ref[...], (_RBP, 128))
        n0 = jnp.reshape(g0_ref[...], (_RBP, 128))
        n1 = jnp.reshape(g1_ref[...], (_RBP, 128))
        n2_ = jnp.reshape(g2_ref[...], (_RBP, 128))
        s_sum = n0 + n1 + n2_
        s_dif = jnp.abs(n2_ - n1) + 2.0 * jnp.abs(n1 - n0)
        s_div = jnp.abs(n0 - f) + jnp.abs(n1 - f) + jnp.abs(n2_ - f)
        z = jnp.concatenate([f, s_sum, s_dif, s_div], axis=1)
        y = lax.dot_general(z, w_ref[...], (((1,), (0,)), ((), ())),
                            preferred_element_type=jnp.float32,
                            precision=_PREC)
        y_ref[...] = y.astype(jnp.bfloat16)
        ids = i * _RBP + lax.broadcasted_iota(jnp.int32, y.shape, 0)
        ym = jnp.where(ids < n2, y, 0.0)

        @pl.when(i == 0)
        def _():
            s1_ref[...] = jnp.zeros_like(s1_ref)
            s2_ref[...] = jnp.zeros_like(s2_ref)

        s1_ref[...] += jnp.sum(ym, axis=0, keepdims=True)
        s2_ref[...] += jnp.sum(ym * ym, axis=0, keepdims=True)

    return pl.pallas_call(
        body,
        grid=(nb,),
        in_specs=[
            pl.BlockSpec((blk,), lambda i: (i,)),
            pl.BlockSpec((blk,), lambda i: (i,)),
            pl.BlockSpec((blk,), lambda i: (i + kstride,)),
            pl.BlockSpec((blk,), lambda i: (i + 2 * kstride,)),
            pl.BlockSpec((512, 128), lambda i: (0, 0)),
        ],
        out_specs=[
            pl.BlockSpec((_RBP, 128), lambda i: (i, 0)),
            pl.BlockSpec((1, 128), lambda i: (0, 0)),
            pl.BlockSpec((1, 128), lambda i: (0, 0)),
        ],
        out_shape=[
            jax.ShapeDtypeStruct((nb * _RBP, 128), jnp.bfloat16),
            jax.ShapeDtypeStruct((1, 128), jnp.float32),
            jax.ShapeDtypeStruct((1, 128), jnp.float32),
        ],
    )(st1d, gath1d, gath1d, gath1d, wbig)


def _norm_relu_conv_stats(y2, a, c, wbd, n2):
    nb = -(-n2 // _RBP2)

    def body(y_ref, a_ref, c_ref, w_ref, y3_ref, s1_ref, s2_ref):
        i = pl.program_id(0)
        y2f = y_ref[...].astype(jnp.float32)
        st1 = jnp.maximum(a_ref[...] * y2f + c_ref[...], 0.0)
        y3 = lax.dot_general(st1, w_ref[...], (((1,), (0,)), ((), ())),
                             preferred_element_type=jnp.float32,
                             precision=_PREC)
        y3_ref[...] = y3.astype(jnp.bfloat16)
        ids = i * _RBP2 + lax.broadcasted_iota(jnp.int32, y3.shape, 0)
        ym = jnp.where(ids < n2, y3, 0.0)

        @pl.when(i == 0)
        def _():
            s1_ref[...] = jnp.zeros_like(s1_ref)
            s2_ref[...] = jnp.zeros_like(s2_ref)

        s1_ref[...] += jnp.sum(ym, axis=0, keepdims=True)
        s2_ref[...] += jnp.sum(ym * ym, axis=0, keepdims=True)

    return pl.pallas_call(
        body,
        grid=(nb,),
        in_specs=[
            pl.BlockSpec((_RBP2, 128), lambda i: (i, 0)),
            pl.BlockSpec((1, 128), lambda i: (0, 0)),
            pl.BlockSpec((1, 128), lambda i: (0, 0)),
            pl.BlockSpec((128, 128), lambda i: (0, 0)),
        ],
        out_specs=[
            pl.BlockSpec((_RBP2, 128), lambda i: (i, 0)),
            pl.BlockSpec((1, 128), lambda i: (0, 0)),
            pl.BlockSpec((1, 128), lambda i: (0, 0)),
        ],
        out_shape=[
            jax.ShapeDtypeStruct((nb * _RBP2, 128), jnp.bfloat16),
            jax.ShapeDtypeStruct((1, 128), jnp.float32),
            jax.ShapeDtypeStruct((1, 128), jnp.float32),
        ],
    )(y2, a, c, wbd)


def _norm_relu_packed(y, a, c, n2):
    nb = -(-n2 // _RBP2)

    def body(y_ref, a_ref, c_ref, o_ref):
        y3 = y_ref[...].astype(jnp.float32)
        o_ref[...] = jnp.maximum(a_ref[...] * y3 + c_ref[...], 0.0)

    return pl.pallas_call(
        body,
        grid=(nb,),
        in_specs=[
            pl.BlockSpec((_RBP2, 128), lambda i: (i, 0)),
            pl.BlockSpec((1, 128), lambda i: (0, 0)),
            pl.BlockSpec((1, 128), lambda i: (0, 0)),
        ],
        out_specs=pl.BlockSpec((_RBP2, 128), lambda i: (i, 0)),
        out_shape=jax.ShapeDtypeStruct((n2, 128), jnp.float32),
    )(y, a, c)


def _bn_coefs(s1, s2, gamma, beta, n):
    m = s1 / n
    v = s2 / n - m * m
    a = gamma * lax.rsqrt(v + _EPS)
    return a, beta - a * m


def kernel(spatial_fea, structural_fea, tex_fea, neighbor_index,
           W1, b1, g1, be1, W2, b2, g2, be2, W3, b3, g3, be3):
    B, c_sp, N = spatial_fea.shape
    c_st = structural_fea.shape[1]
    del b1, b2, b3

    sp = spatial_fea[0]
    st = structural_fea[0]
    tex = tex_fea[0]

    unit = 32 * _GRP // 3 if (32 * _GRP) % 3 == 0 else 32 * _GRP
    npad = -(-N // unit) * unit
    st1d = st.T.reshape(-1)
    stT = st1d.reshape(N, 64)
    zpad = jnp.zeros((npad - N,), jnp.int32)
    nix = neighbor_index[0]
    idx_flat = jnp.concatenate(
        [nix[:, 0], zpad, nix[:, 1], zpad, nix[:, 2], zpad])
    gath = _gather_rows(stT, idx_flat)
    gath1d = gath.reshape(-1)

    y1, s1, q1 = _conv_stats_cn(sp, st, tex, W1[:, :c_sp],
                                W1[:, c_sp:c_sp + c_st],
                                W1[:, c_sp + c_st:], N)
    a1, c1 = _bn_coefs(s1[:, 0], q1[:, 0], g1, be1, N)
    out_sp = _norm_relu_cn(y1, a1.reshape(-1, 1), c1.reshape(-1, 1), N)

    gath1d, out_sp = lax.optimization_barrier((gath1d, out_sp))

    n2 = N // 2
    eye2 = jnp.eye(2, dtype=jnp.float32)
    W2T = W2.T
    wbig = jnp.concatenate(
        [jnp.kron(eye2, W2T[k * c_st:(k + 1) * c_st]) for k in range(4)],
        axis=0)
    y2, s2, q2 = _neighbor_conv_stats(st1d, gath1d, wbig, n2, npad)
    a2, c2 = _bn_coefs(s2[0, :64] + s2[0, 64:], q2[0, :64] + q2[0, 64:],
                       g2, be2, N)
    a2p = jnp.tile(a2, 2).reshape(1, 128)
    c2p = jnp.tile(c2, 2).reshape(1, 128)
    w3bd = jnp.kron(eye2, W3.T)
    y3, s3, q3 = _norm_relu_conv_stats(y2, a2p, c2p, w3bd, n2)
    a3, c3 = _bn_coefs(s3[0, :64] + s3[0, 64:], q3[0, :64] + q3[0, 64:],
                       g3, be3, N)
    outP = _norm_relu_packed(y3, jnp.tile(a3, 2).reshape(1, 128),
                             jnp.tile(c3, 2).reshape(1, 128), n2)
    out_st = outP.reshape(N, 64).T

    return out_sp[None], out_st[None]


# speedup vs baseline: 1.0521x; 1.0045x over previous
import functools

import jax
import jax.numpy as jnp
from jax import lax
from jax.experimental import pallas as pl
from jax.experimental.pallas import tpu as pltpu
from jax.experimental.pallas import tpu_sc as plsc

_EPS = 1e-5
_LB = 8192
_RBP = 2560
_RBP2 = 4096
_CH = 128
_CPG = 5
_GRP = _CH * _CPG
_NBUF = 3
_PREC = lax.Precision.DEFAULT


def _gather_rows(table, idx_flat):
    G = idx_flat.shape[0]
    info = plsc.get_sparse_core_info()
    NC, NS = info.num_cores, info.num_subcores
    NW = NC * NS
    per_w = G // NW
    n_groups = per_w // _GRP
    assert per_w % _GRP == 0 and G % NW == 0
    D = table.shape[1]
    mesh = plsc.VectorSubcoreMesh(core_axis_name="c", subcore_axis_name="s")

    @functools.partial(
        pl.kernel,
        mesh=mesh,
        compiler_params=pltpu.CompilerParams(use_tc_tiling_on_sc=False),
        cost_estimate=pl.CostEstimate(
            flops=0, bytes_accessed=int(G * D * 4 * 2), transcendentals=0),
        out_type=jax.ShapeDtypeStruct((G, D), jnp.float32),
        scratch_types=[
            pltpu.VMEM((_GRP,), jnp.int32),
            pltpu.VMEM((_GRP,), jnp.int32),
            pltpu.VMEM((_GRP,), jnp.int32),
            pltpu.VMEM((_GRP, D), jnp.float32),
            pltpu.VMEM((_GRP, D), jnp.float32),
            pltpu.VMEM((_GRP, D), jnp.float32),
            pltpu.SemaphoreType.DMA,
            pltpu.SemaphoreType.DMA,
            pltpu.SemaphoreType.DMA,
            pltpu.SemaphoreType.DMA,
            pltpu.SemaphoreType.DMA,
            pltpu.SemaphoreType.DMA,
        ],
    )
    def k(table_hbm, idx_hbm, out_hbm, i0, i1, i2, r0, r1, r2,
          gs0, gs1, gs2, ss0, ss1, ss2):
        idxs = [i0, i1, i2]
        rows = [r0, r1, r2]
        gsem = [gs0, gs1, gs2]
        ssem = [ss0, ss1, ss2]
        wid = lax.axis_index("s") * NC + lax.axis_index("c")
        base_w = wid * per_w
        gcopies = [None] * _NBUF
        stores = [None] * _NBUF
        for g in range(n_groups):
            b = g % _NBUF
            if g >= _NBUF:
                stores[b].wait()
            pltpu.sync_copy(idx_hbm.at[pl.ds(base_w + g * _GRP, _GRP)],
                            idxs[b])
            cs = []
            for j in range(_CPG):
                cs.append(pltpu.async_copy(
                    table_hbm.at[idxs[b].at[pl.ds(j * _CH, _CH)]],
                    rows[b].at[pl.ds(j * _CH, _CH)], gsem[b]))
            gcopies[b] = cs
            if g >= 1:
                pb = (g - 1) % _NBUF
                for c in gcopies[pb]:
                    c.wait()
                stores[pb] = pltpu.async_copy(
                    rows[pb],
                    out_hbm.at[pl.ds(base_w + (g - 1) * _GRP, _GRP)],
                    ssem[pb])
        lb = (n_groups - 1) % _NBUF
        for c in gcopies[lb]:
            c.wait()
        stores[lb] = pltpu.async_copy(
            rows[lb],
            out_hbm.at[pl.ds(base_w + (n_groups - 1) * _GRP, _GRP)],
            ssem[lb])
        for b in range(_NBUF):
            stores[b].wait()

    return k(table, idx_flat)


def _conv_stats_cn(sp, st, tex, w_sp, w_st, w_tx, n):
    nb = -(-n // _LB)
    co = w_sp.shape[0]

    def body(sp_ref, st_ref, tex_ref, wa_ref, wb_ref, wc_ref,
             y_ref, s1_ref, s2_ref):
        i = pl.program_id(0)
        dn = (((1,), (0,)), ((), ()))
        y = lax.dot_general(wa_ref[...], sp_ref[...], dn,
                            preferred_element_type=jnp.float32,
                            precision=_PREC)
        y += lax.dot_general(wb_ref[...], st_ref[...], dn,
                             preferred_element_type=jnp.float32,
                             precision=_PREC)
        y += lax.dot_general(wc_ref[...], tex_ref[...], dn,
                             preferred_element_type=jnp.float32,
                             precision=_PREC)
        y_ref[...] = y.astype(jnp.bfloat16)
        ids = i * _LB + lax.broadcasted_iota(jnp.int32, y.shape, 1)
        ym = jnp.where(ids < n, y, 0.0)

        @pl.when(i == 0)
        def _():
            s1_ref[...] = jnp.zeros_like(s1_ref)
            s2_ref[...] = jnp.zeros_like(s2_ref)

        s1_ref[...] += jnp.sum(ym, axis=1, keepdims=True)
        s2_ref[...] += jnp.sum(ym * ym, axis=1, keepdims=True)

    c_sp, c_st, c_tx = sp.shape[0], st.shape[0], tex.shape[0]
    return pl.pallas_call(
        body,
        grid=(nb,),
        in_specs=[
            pl.BlockSpec((c_sp, _LB), lambda i: (0, i)),
            pl.BlockSpec((c_st, _LB), lambda i: (0, i)),
            pl.BlockSpec((c_tx, _LB), lambda i: (0, i)),
            pl.BlockSpec((co, c_sp), lambda i: (0, 0)),
            pl.BlockSpec((co, c_st), lambda i: (0, 0)),
            pl.BlockSpec((co, c_tx), lambda i: (0, 0)),
        ],
        out_specs=[
            pl.BlockSpec((co, _LB), lambda i: (0, i)),
            pl.BlockSpec((co, 1), lambda i: (0, 0)),
            pl.BlockSpec((co, 1), lambda i: (0, 0)),
        ],
        out_shape=[
            jax.ShapeDtypeStruct((co, nb * _LB), jnp.bfloat16),
            jax.ShapeDtypeStruct((co, 1), jnp.float32),
            jax.ShapeDtypeStruct((co, 1), jnp.float32),
        ],
    )(sp, st, tex, w_sp, w_st, w_tx)


def _norm_relu_cn(y, a, c, n):
    nb = -(-n // _LB)
    co = y.shape[0]

    def body(y_ref, a_ref, c_ref, o_ref):
        y = y_ref[...].astype(jnp.float32)
        o_ref[...] = jnp.maximum(a_ref[...] * y + c_ref[...], 0.0)

    return pl.pallas_call(
        body,
        grid=(nb,),
        in_specs=[
            pl.BlockSpec((co, _LB), lambda i: (0, i)),
            pl.BlockSpec((co, 1), lambda i: (0, 0)),
            pl.BlockSpec((co, 1), lambda i: (0, 0)),
        ],
        out_specs=pl.BlockSpec((co, _LB), lambda i: (0, i)),
        out_shape=jax.ShapeDtypeStruct((co, n), jnp.float32),
    )(y, a, c)


def _neighbor_conv_stats(st1d, gath1d, wbig, n2, npad):
    nb = -(-n2 // _RBP)
    kstride = npad * 64 // (_RBP * 128)
    blk = _RBP * 128

    def body(f_ref, g0_ref, g1_ref, g2_ref, w_ref, y_ref, s1_ref, s2_ref):
        i = pl.program_id(0)
        f = jnp.reshape(f_ref[...], (_RBP, 128))
        n0 = jnp.reshape(g0_ref[...], (_RBP, 128))
        n1 = jnp.reshape(g1_ref[...], (_RBP, 128))
        n2_ = jnp.reshape(g2_ref[...], (_RBP, 128))
        s_sum = n0 + n1 + n2_
        s_dif = jnp.abs(n2_ - n1) + 2.0 * jnp.abs(n1 - n0)
        s_div = jnp.abs(n0 - f) + jnp.abs(n1 - f) + jnp.abs(n2_ - f)
        z = jnp.concatenate([f, s_sum, s_dif, s_div], axis=1)
        y = lax.dot_general(z, w_ref[...], (((1,), (0,)), ((), ())),
                            preferred_element_type=jnp.float32,
                            precision=_PREC)
        y_ref[...] = y.astype(jnp.bfloat16)
        ids = i * _RBP + lax.broadcasted_iota(jnp.int32, y.shape, 0)
        ym = jnp.where(ids < n2, y, 0.0)

        @pl.when(i == 0)
        def _():
            s1_ref[...] = jnp.zeros_like(s1_ref)
            s2_ref[...] = jnp.zeros_like(s2_ref)

        s1_ref[...] += jnp.sum(ym, axis=0, keepdims=True)
        s2_ref[...] += jnp.sum(ym * ym, axis=0, keepdims=True)

    return pl.pallas_call(
        body,
        grid=(nb,),
        in_specs=[
            pl.BlockSpec((blk,), lambda i: (i,)),
            pl.BlockSpec((blk,), lambda i: (i,)),
            pl.BlockSpec((blk,), lambda i: (i + kstride,)),
            pl.BlockSpec((blk,), lambda i: (i + 2 * kstride,)),
            pl.BlockSpec((512, 128), lambda i: (0, 0)),
        ],
        out_specs=[
            pl.BlockSpec((_RBP, 128), lambda i: (i, 0)),
            pl.BlockSpec((1, 128), lambda i: (0, 0)),
            pl.BlockSpec((1, 128), lambda i: (0, 0)),
        ],
        out_shape=[
            jax.ShapeDtypeStruct((nb * _RBP, 128), jnp.bfloat16),
            jax.ShapeDtypeStruct((1, 128), jnp.float32),
            jax.ShapeDtypeStruct((1, 128), jnp.float32),
        ],
    )(st1d, gath1d, gath1d, gath1d, wbig)


def _norm_relu_conv_stats(y2, a, c, wbd, n2):
    nb = -(-n2 // _RBP2)

    def body(y_ref, a_ref, c_ref, w_ref, y3_ref, s1_ref, s2_ref):
        i = pl.program_id(0)
        y2f = y_ref[...].astype(jnp.float32)
        st1 = jnp.maximum(a_ref[...] * y2f + c_ref[...], 0.0)
        y3 = lax.dot_general(st1, w_ref[...], (((1,), (0,)), ((), ())),
                             preferred_element_type=jnp.float32,
                             precision=_PREC)
        y3_ref[...] = y3.astype(jnp.bfloat16)
        ids = i * _RBP2 + lax.broadcasted_iota(jnp.int32, y3.shape, 0)
        ym = jnp.where(ids < n2, y3, 0.0)

        @pl.when(i == 0)
        def _():
            s1_ref[...] = jnp.zeros_like(s1_ref)
            s2_ref[...] = jnp.zeros_like(s2_ref)

        s1_ref[...] += jnp.sum(ym, axis=0, keepdims=True)
        s2_ref[...] += jnp.sum(ym * ym, axis=0, keepdims=True)

    return pl.pallas_call(
        body,
        grid=(nb,),
        in_specs=[
            pl.BlockSpec((_RBP2, 128), lambda i: (i, 0)),
            pl.BlockSpec((1, 128), lambda i: (0, 0)),
            pl.BlockSpec((1, 128), lambda i: (0, 0)),
            pl.BlockSpec((128, 128), lambda i: (0, 0)),
        ],
        out_specs=[
            pl.BlockSpec((_RBP2, 128), lambda i: (i, 0)),
            pl.BlockSpec((1, 128), lambda i: (0, 0)),
            pl.BlockSpec((1, 128), lambda i: (0, 0)),
        ],
        out_shape=[
            jax.ShapeDtypeStruct((nb * _RBP2, 128), jnp.bfloat16),
            jax.ShapeDtypeStruct((1, 128), jnp.float32),
            jax.ShapeDtypeStruct((1, 128), jnp.float32),
        ],
    )(y2, a, c, wbd)


def _norm_relu_packed(y, a, c, n2):
    nb = -(-n2 // _RBP2)

    def body(y_ref, a_ref, c_ref, o_ref):
        y3 = y_ref[...].astype(jnp.float32)
        o_ref[...] = jnp.maximum(a_ref[...] * y3 + c_ref[...], 0.0)

    return pl.pallas_call(
        body,
        grid=(nb,),
        in_specs=[
            pl.BlockSpec((_RBP2, 128), lambda i: (i, 0)),
            pl.BlockSpec((1, 128), lambda i: (0, 0)),
            pl.BlockSpec((1, 128), lambda i: (0, 0)),
        ],
        out_specs=pl.BlockSpec((_RBP2, 128), lambda i: (i, 0)),
        out_shape=jax.ShapeDtypeStruct((n2, 128), jnp.float32),
    )(y, a, c)


def _bn_coefs(s1, s2, gamma, beta, n):
    m = s1 / n
    v = s2 / n - m * m
    a = gamma * lax.rsqrt(v + _EPS)
    return a, beta - a * m


def kernel(spatial_fea, structural_fea, tex_fea, neighbor_index,
           W1, b1, g1, be1, W2, b2, g2, be2, W3, b3, g3, be3):
    B, c_sp, N = spatial_fea.shape
    c_st = structural_fea.shape[1]
    del b1, b2, b3

    sp = spatial_fea[0]
    st = structural_fea[0]
    tex = tex_fea[0]

    unit = 32 * _GRP // 3 if (32 * _GRP) % 3 == 0 else 32 * _GRP
    npad = -(-N // unit) * unit
    st1d = st.T.reshape(-1)
    stT = st1d.reshape(N, 64)
    zpad = jnp.zeros((npad - N,), jnp.int32)
    nix = neighbor_index[0]
    idx_flat = jnp.concatenate(
        [nix[:, 0], zpad, nix[:, 1], zpad, nix[:, 2], zpad])
    gath = _gather_rows(stT, idx_flat)
    gath1d = gath.reshape(-1)

    y1, s1, q1 = _conv_stats_cn(sp, st, tex, W1[:, :c_sp],
                                W1[:, c_sp:c_sp + c_st],
                                W1[:, c_sp + c_st:], N)
    a1, c1 = _bn_coefs(s1[:, 0], q1[:, 0], g1, be1, N)
    out_sp = _norm_relu_cn(y1, a1.reshape(-1, 1), c1.reshape(-1, 1), N)

    gath1d, out_sp = lax.optimization_barrier((gath1d, out_sp))

    n2 = N // 2
    eye2 = jnp.eye(2, dtype=jnp.float32)
    W2T = W2.T
    wbig = jnp.concatenate(
        [jnp.kron(eye2, W2T[k * c_st:(k + 1) * c_st]) for k in range(4)],
        axis=0)
    y2, s2, q2 = _neighbor_conv_stats(st1d, gath1d, wbig, n2, npad)
    a2, c2 = _bn_coefs(s2[0, :64] + s2[0, 64:], q2[0, :64] + q2[0, 64:],
                       g2, be2, N)
    a2p = jnp.tile(a2, 2).reshape(1, 128)
    c2p = jnp.tile(c2, 2).reshape(1, 128)
    w3bd = jnp.kron(eye2, W3.T)
    y3, s3, q3 = _norm_relu_conv_stats(y2, a2p, c2p, w3bd, n2)
    a3, c3 = _bn_coefs(s3[0, :64] + s3[0, 64:], q3[0, :64] + q3[0, 64:],
                       g3, be3, N)
    outP = _norm_relu_packed(y3, jnp.tile(a3, 2).reshape(1, 128),
                             jnp.tile(c3, 2).reshape(1, 128), n2)
    out_st = outP.reshape(N, 64).T

    return out_sp[None], out_st[None]


# speedup vs baseline: 1.0641x; 1.0114x over previous
import functools

import jax
import jax.numpy as jnp
from jax import lax
from jax.experimental import pallas as pl
from jax.experimental.pallas import tpu as pltpu
from jax.experimental.pallas import tpu_sc as plsc

_EPS = 1e-5
_LB = 8192
_RBP = 2560
_RBP2 = 4096
_CH = 128
_CPG = 5
_GRP = _CH * _CPG
_NBUF = 3
_PREC = lax.Precision.DEFAULT


def _gather_rows(table, idx_flat):
    G = idx_flat.shape[0]
    info = plsc.get_sparse_core_info()
    NC, NS = info.num_cores, info.num_subcores
    NW = NC * NS
    per_w = G // NW
    n_groups = per_w // _GRP
    assert per_w % _GRP == 0 and G % NW == 0
    D = table.shape[1]
    mesh = plsc.VectorSubcoreMesh(core_axis_name="c", subcore_axis_name="s")

    @functools.partial(
        pl.kernel,
        mesh=mesh,
        compiler_params=pltpu.CompilerParams(use_tc_tiling_on_sc=False),
        cost_estimate=pl.CostEstimate(
            flops=0, bytes_accessed=int(G * D * 4 * 2), transcendentals=0),
        out_type=jax.ShapeDtypeStruct((G, D), jnp.float32),
        scratch_types=[
            pltpu.VMEM((_GRP,), jnp.int32),
            pltpu.VMEM((_GRP,), jnp.int32),
            pltpu.VMEM((_GRP,), jnp.int32),
            pltpu.VMEM((_GRP, D), jnp.float32),
            pltpu.VMEM((_GRP, D), jnp.float32),
            pltpu.VMEM((_GRP, D), jnp.float32),
            pltpu.SemaphoreType.DMA,
            pltpu.SemaphoreType.DMA,
            pltpu.SemaphoreType.DMA,
            pltpu.SemaphoreType.DMA,
            pltpu.SemaphoreType.DMA,
            pltpu.SemaphoreType.DMA,
        ],
    )
    def k(table_hbm, idx_hbm, out_hbm, i0, i1, i2, r0, r1, r2,
          gs0, gs1, gs2, ss0, ss1, ss2):
        idxs = [i0, i1, i2]
        rows = [r0, r1, r2]
        gsem = [gs0, gs1, gs2]
        ssem = [ss0, ss1, ss2]
        wid = lax.axis_index("s") * NC + lax.axis_index("c")
        base_w = wid * per_w
        gcopies = [None] * _NBUF
        stores = [None] * _NBUF
        for g in range(n_groups):
            b = g % _NBUF
            if g >= _NBUF:
                stores[b].wait()
            pltpu.sync_copy(idx_hbm.at[pl.ds(base_w + g * _GRP, _GRP)],
                            idxs[b])
            cs = []
            for j in range(_CPG):
                cs.append(pltpu.async_copy(
                    table_hbm.at[idxs[b].at[pl.ds(j * _CH, _CH)]],
                    rows[b].at[pl.ds(j * _CH, _CH)], gsem[b]))
            gcopies[b] = cs
            if g >= 1:
                pb = (g - 1) % _NBUF
                for c in gcopies[pb]:
                    c.wait()
                stores[pb] = pltpu.async_copy(
                    rows[pb],
                    out_hbm.at[pl.ds(base_w + (g - 1) * _GRP, _GRP)],
                    ssem[pb])
        lb = (n_groups - 1) % _NBUF
        for c in gcopies[lb]:
            c.wait()
        stores[lb] = pltpu.async_copy(
            rows[lb],
            out_hbm.at[pl.ds(base_w + (n_groups - 1) * _GRP, _GRP)],
            ssem[lb])
        for b in range(_NBUF):
            stores[b].wait()

    return k(table, idx_flat)


def _conv_stats_cn(sp, st, tex, w_sp, w_st, w_tx, n):
    nb = -(-n // _LB)
    co = w_sp.shape[0]

    def body(sp_ref, st_ref, tex_ref, wa_ref, wb_ref, wc_ref,
             y_ref, s1_ref, s2_ref):
        i = pl.program_id(0)
        dn = (((1,), (0,)), ((), ()))
        y = lax.dot_general(wa_ref[...], sp_ref[...], dn,
                            preferred_element_type=jnp.float32,
                            precision=_PREC)
        y += lax.dot_general(wb_ref[...], st_ref[...], dn,
                             preferred_element_type=jnp.float32,
                             precision=_PREC)
        y += lax.dot_general(wc_ref[...], tex_ref[...], dn,
                             preferred_element_type=jnp.float32,
                             precision=_PREC)
        y_ref[...] = y.astype(jnp.bfloat16)
        ids = i * _LB + lax.broadcasted_iota(jnp.int32, y.shape, 1)
        ym = jnp.where(ids < n, y, 0.0)

        @pl.when(i == 0)
        def _():
            s1_ref[...] = jnp.zeros_like(s1_ref)
            s2_ref[...] = jnp.zeros_like(s2_ref)

        s1_ref[...] += jnp.sum(ym, axis=1, keepdims=True)
        s2_ref[...] += jnp.sum(ym * ym, axis=1, keepdims=True)

    c_sp, c_st, c_tx = sp.shape[0], st.shape[0], tex.shape[0]
    return pl.pallas_call(
        body,
        grid=(nb,),
        in_specs=[
            pl.BlockSpec((c_sp, _LB), lambda i: (0, i)),
            pl.BlockSpec((c_st, _LB), lambda i: (0, i)),
            pl.BlockSpec((c_tx, _LB), lambda i: (0, i)),
            pl.BlockSpec((co, c_sp), lambda i: (0, 0)),
            pl.BlockSpec((co, c_st), lambda i: (0, 0)),
            pl.BlockSpec((co, c_tx), lambda i: (0, 0)),
        ],
        out_specs=[
            pl.BlockSpec((co, _LB), lambda i: (0, i)),
            pl.BlockSpec((co, 1), lambda i: (0, 0)),
            pl.BlockSpec((co, 1), lambda i: (0, 0)),
        ],
        out_shape=[
            jax.ShapeDtypeStruct((co, nb * _LB), jnp.bfloat16),
            jax.ShapeDtypeStruct((co, 1), jnp.float32),
            jax.ShapeDtypeStruct((co, 1), jnp.float32),
        ],
    )(sp, st, tex, w_sp, w_st, w_tx)


def _norm_relu_cn(y, a, c, n):
    nb = -(-n // _LB)
    co = y.shape[0]

    def body(y_ref, a_ref, c_ref, o_ref):
        y = y_ref[...].astype(jnp.float32)
        o_ref[...] = jnp.maximum(a_ref[...] * y + c_ref[...], 0.0)

    return pl.pallas_call(
        body,
        grid=(nb,),
        in_specs=[
            pl.BlockSpec((co, _LB), lambda i: (0, i)),
            pl.BlockSpec((co, 1), lambda i: (0, 0)),
            pl.BlockSpec((co, 1), lambda i: (0, 0)),
        ],
        out_specs=pl.BlockSpec((co, _LB), lambda i: (0, i)),
        out_shape=jax.ShapeDtypeStruct((co, n), jnp.float32),
    )(y, a, c)


def _neighbor_conv_stats(st1d, gath1d, wbig, n2, npad):
    nb = -(-n2 // _RBP)
    kstride = npad * 64 // (_RBP * 128)
    blk = _RBP * 128

    def body(f_ref, g0_ref, g1_ref, g2_ref, w_ref, y_ref, s1_ref, s2_ref):
        i = pl.program_id(0)
        f = jnp.reshape(f_ref[...], (_RBP, 128))
        n0 = jnp.reshape(g0_ref[...], (_RBP, 128))
        n1 = jnp.reshape(g1_ref[...], (_RBP, 128))
        n2_ = jnp.reshape(g2_ref[...], (_RBP, 128))
        s_sum = n0 + n1 + n2_
        s_dif = jnp.abs(n2_ - n1) + 2.0 * jnp.abs(n1 - n0)
        s_div = jnp.abs(n0 - f) + jnp.abs(n1 - f) + jnp.abs(n2_ - f)
        z = jnp.concatenate([f, s_sum, s_dif, s_div], axis=1)
        y = lax.dot_general(z, w_ref[...], (((1,), (0,)), ((), ())),
                            preferred_element_type=jnp.float32,
                            precision=_PREC)
        y_ref[...] = y.astype(jnp.bfloat16)
        ids = i * _RBP + lax.broadcasted_iota(jnp.int32, y.shape, 0)
        ym = jnp.where(ids < n2, y, 0.0)

        @pl.when(i == 0)
        def _():
            s1_ref[...] = jnp.zeros_like(s1_ref)
            s2_ref[...] = jnp.zeros_like(s2_ref)

        s1_ref[...] += jnp.sum(ym, axis=0, keepdims=True)
        s2_ref[...] += jnp.sum(ym * ym, axis=0, keepdims=True)

    return pl.pallas_call(
        body,
        grid=(nb,),
        in_specs=[
            pl.BlockSpec((blk,), lambda i: (i,)),
            pl.BlockSpec((blk,), lambda i: (i,)),
            pl.BlockSpec((blk,), lambda i: (i + kstride,)),
            pl.BlockSpec((blk,), lambda i: (i + 2 * kstride,)),
            pl.BlockSpec((512, 128), lambda i: (0, 0)),
        ],
        out_specs=[
            pl.BlockSpec((_RBP, 128), lambda i: (i, 0)),
            pl.BlockSpec((1, 128), lambda i: (0, 0)),
            pl.BlockSpec((1, 128), lambda i: (0, 0)),
        ],
        out_shape=[
            jax.ShapeDtypeStruct((nb * _RBP, 128), jnp.bfloat16),
            jax.ShapeDtypeStruct((1, 128), jnp.float32),
            jax.ShapeDtypeStruct((1, 128), jnp.float32),
        ],
    )(st1d, gath1d, gath1d, gath1d, wbig)


def _norm_relu_conv_stats(y2, a, c, wbd, n2):
    nb = -(-n2 // _RBP2)

    def body(y_ref, a_ref, c_ref, w_ref, y3_ref, s1_ref, s2_ref):
        i = pl.program_id(0)
        y2f = y_ref[...].astype(jnp.float32)
        st1 = jnp.maximum(a_ref[...] * y2f + c_ref[...], 0.0)
        y3 = lax.dot_general(st1, w_ref[...], (((1,), (0,)), ((), ())),
                             preferred_element_type=jnp.float32,
                             precision=_PREC)
        y3_ref[...] = y3.astype(jnp.bfloat16)
        ids = i * _RBP2 + lax.broadcasted_iota(jnp.int32, y3.shape, 0)
        ym = jnp.where(ids < n2, y3, 0.0)

        @pl.when(i == 0)
        def _():
            s1_ref[...] = jnp.zeros_like(s1_ref)
            s2_ref[...] = jnp.zeros_like(s2_ref)

        s1_ref[...] += jnp.sum(ym, axis=0, keepdims=True)
        s2_ref[...] += jnp.sum(ym * ym, axis=0, keepdims=True)

    return pl.pallas_call(
        body,
        grid=(nb,),
        in_specs=[
            pl.BlockSpec((_RBP2, 128), lambda i: (i, 0)),
            pl.BlockSpec((1, 128), lambda i: (0, 0)),
            pl.BlockSpec((1, 128), lambda i: (0, 0)),
            pl.BlockSpec((128, 128), lambda i: (0, 0)),
        ],
        out_specs=[
            pl.BlockSpec((_RBP2, 128), lambda i: (i, 0)),
            pl.BlockSpec((1, 128), lambda i: (0, 0)),
            pl.BlockSpec((1, 128), lambda i: (0, 0)),
        ],
        out_shape=[
            jax.ShapeDtypeStruct((nb * _RBP2, 128), jnp.bfloat16),
            jax.ShapeDtypeStruct((1, 128), jnp.float32),
            jax.ShapeDtypeStruct((1, 128), jnp.float32),
        ],
    )(y2, a, c, wbd)


def _bn_coefs(s1, s2, gamma, beta, n):
    m = s1 / n
    v = s2 / n - m * m
    a = gamma * lax.rsqrt(v + _EPS)
    return a, beta - a * m


def kernel(spatial_fea, structural_fea, tex_fea, neighbor_index,
           W1, b1, g1, be1, W2, b2, g2, be2, W3, b3, g3, be3):
    B, c_sp, N = spatial_fea.shape
    c_st = structural_fea.shape[1]
    del b1, b2, b3

    sp = spatial_fea[0]
    st = structural_fea[0]
    tex = tex_fea[0]

    unit = 32 * _GRP // 3 if (32 * _GRP) % 3 == 0 else 32 * _GRP
    npad = -(-N // unit) * unit
    st1d = st.T.reshape(-1)
    stT = st1d.reshape(N, 64)
    zpad = jnp.zeros((npad - N,), jnp.int32)
    nix = neighbor_index[0]
    idx_flat = jnp.concatenate(
        [nix[:, 0], zpad, nix[:, 1], zpad, nix[:, 2], zpad])
    gath = _gather_rows(stT, idx_flat)
    gath1d = gath.reshape(-1)

    y1, s1, q1 = _conv_stats_cn(sp, st, tex, W1[:, :c_sp],
                                W1[:, c_sp:c_sp + c_st],
                                W1[:, c_sp + c_st:], N)
    a1, c1 = _bn_coefs(s1[:, 0], q1[:, 0], g1, be1, N)
    out_sp = _norm_relu_cn(y1, a1.reshape(-1, 1), c1.reshape(-1, 1), N)

    gath1d, out_sp = lax.optimization_barrier((gath1d, out_sp))

    n2 = N // 2
    eye2 = jnp.eye(2, dtype=jnp.float32)
    W2T = W2.T
    wbig = jnp.concatenate(
        [jnp.kron(eye2, W2T[k * c_st:(k + 1) * c_st]) for k in range(4)],
        axis=0)
    y2, s2, q2 = _neighbor_conv_stats(st1d, gath1d, wbig, n2, npad)
    a2, c2 = _bn_coefs(s2[0, :64] + s2[0, 64:], q2[0, :64] + q2[0, 64:],
                       g2, be2, N)
    a2p = jnp.tile(a2, 2).reshape(1, 128)
    c2p = jnp.tile(c2, 2).reshape(1, 128)
    w3bd = jnp.kron(eye2, W3.T)
    y3, s3, q3 = _norm_relu_conv_stats(y2, a2p, c2p, w3bd, n2)
    a3, c3 = _bn_coefs(s3[0, :64] + s3[0, 64:], q3[0, :64] + q3[0, 64:],
                       g3, be3, N)
    y3t = y3[:n2].reshape(N, 64).T
    out_st = _norm_relu_cn(y3t, a3.reshape(-1, 1), c3.reshape(-1, 1), N)

    return out_sp[None], out_st[None]


# speedup vs baseline: 1.0641x; 1.0000x over previous
import functools

import jax
import jax.numpy as jnp
from jax import lax
from jax.experimental import pallas as pl
from jax.experimental.pallas import tpu as pltpu
from jax.experimental.pallas import tpu_sc as plsc

_EPS = 1e-5
_LB = 8192
_RBP = 2560
_RBP2 = 4096
_CH = 128
_CPG = 5
_GRP = _CH * _CPG
_NBUF = 3
_PREC = lax.Precision.DEFAULT


def _gather_rows(table, idx_flat):
    G = idx_flat.shape[0]
    info = plsc.get_sparse_core_info()
    NC, NS = info.num_cores, info.num_subcores
    NW = NC * NS
    per_w = G // NW
    n_groups = per_w // _GRP
    assert per_w % _GRP == 0 and G % NW == 0
    D = table.shape[1]
    mesh = plsc.VectorSubcoreMesh(core_axis_name="c", subcore_axis_name="s")

    @functools.partial(
        pl.kernel,
        mesh=mesh,
        compiler_params=pltpu.CompilerParams(use_tc_tiling_on_sc=False),
        cost_estimate=pl.CostEstimate(
            flops=0, bytes_accessed=int(G * D * 4 * 2), transcendentals=0),
        out_type=jax.ShapeDtypeStruct((G, D), jnp.float32),
        scratch_types=[
            pltpu.VMEM((_GRP,), jnp.int32),
            pltpu.VMEM((_GRP,), jnp.int32),
            pltpu.VMEM((_GRP,), jnp.int32),
            pltpu.VMEM((_GRP, D), jnp.float32),
            pltpu.VMEM((_GRP, D), jnp.float32),
            pltpu.VMEM((_GRP, D), jnp.float32),
            pltpu.SemaphoreType.DMA,
            pltpu.SemaphoreType.DMA,
            pltpu.SemaphoreType.DMA,
            pltpu.SemaphoreType.DMA,
            pltpu.SemaphoreType.DMA,
            pltpu.SemaphoreType.DMA,
            pltpu.SemaphoreType.DMA,
            pltpu.SemaphoreType.DMA,
            pltpu.SemaphoreType.DMA,
        ],
    )
    def k(table_hbm, idx_hbm, out_hbm, i0, i1, i2, r0, r1, r2,
          gs0, gs1, gs2, ss0, ss1, ss2, is0, is1, is2):
        idxs = [i0, i1, i2]
        rows = [r0, r1, r2]
        gsem = [gs0, gs1, gs2]
        ssem = [ss0, ss1, ss2]
        isem = [is0, is1, is2]
        wid = lax.axis_index("s") * NC + lax.axis_index("c")
        base_w = wid * per_w
        gcopies = [None] * _NBUF
        stores = [None] * _NBUF
        icopies = [None] * _NBUF
        for g in range(n_groups):
            b = g % _NBUF
            if g >= _NBUF:
                stores[b].wait()
            if g == 0:
                pltpu.sync_copy(idx_hbm.at[pl.ds(base_w, _GRP)], idxs[0])
            else:
                icopies[b].wait()
            if g + 1 < n_groups:
                nxt = (g + 1) % _NBUF
                icopies[nxt] = pltpu.async_copy(
                    idx_hbm.at[pl.ds(base_w + (g + 1) * _GRP, _GRP)],
                    idxs[nxt], isem[nxt])
            cs = []
            for j in range(_CPG):
                cs.append(pltpu.async_copy(
                    table_hbm.at[idxs[b].at[pl.ds(j * _CH, _CH)]],
                    rows[b].at[pl.ds(j * _CH, _CH)], gsem[b]))
            gcopies[b] = cs
            if g >= 1:
                pb = (g - 1) % _NBUF
                for c in gcopies[pb]:
                    c.wait()
                stores[pb] = pltpu.async_copy(
                    rows[pb],
                    out_hbm.at[pl.ds(base_w + (g - 1) * _GRP, _GRP)],
                    ssem[pb])
        lb = (n_groups - 1) % _NBUF
        for c in gcopies[lb]:
            c.wait()
        stores[lb] = pltpu.async_copy(
            rows[lb],
            out_hbm.at[pl.ds(base_w + (n_groups - 1) * _GRP, _GRP)],
            ssem[lb])
        for b in range(_NBUF):
            stores[b].wait()

    return k(table, idx_flat)


def _conv_stats_cn(sp, st, tex, w_sp, w_st, w_tx, n):
    nb = -(-n // _LB)
    co = w_sp.shape[0]

    def body(sp_ref, st_ref, tex_ref, wa_ref, wb_ref, wc_ref,
             y_ref, s1_ref, s2_ref):
        i = pl.program_id(0)
        dn = (((1,), (0,)), ((), ()))
        y = lax.dot_general(wa_ref[...], sp_ref[...], dn,
                            preferred_element_type=jnp.float32,
                            precision=_PREC)
        y += lax.dot_general(wb_ref[...], st_ref[...], dn,
                             preferred_element_type=jnp.float32,
                             precision=_PREC)
        y += lax.dot_general(wc_ref[...], tex_ref[...], dn,
                             preferred_element_type=jnp.float32,
                             precision=_PREC)
        y_ref[...] = y.astype(jnp.bfloat16)
        ids = i * _LB + lax.broadcasted_iota(jnp.int32, y.shape, 1)
        ym = jnp.where(ids < n, y, 0.0)

        @pl.when(i == 0)
        def _():
            s1_ref[...] = jnp.zeros_like(s1_ref)
            s2_ref[...] = jnp.zeros_like(s2_ref)

        s1_ref[...] += jnp.sum(ym, axis=1, keepdims=True)
        s2_ref[...] += jnp.sum(ym * ym, axis=1, keepdims=True)

    c_sp, c_st, c_tx = sp.shape[0], st.shape[0], tex.shape[0]
    return pl.pallas_call(
        body,
        grid=(nb,),
        in_specs=[
            pl.BlockSpec((c_sp, _LB), lambda i: (0, i)),
            pl.BlockSpec((c_st, _LB), lambda i: (0, i)),
            pl.BlockSpec((c_tx, _LB), lambda i: (0, i)),
            pl.BlockSpec((co, c_sp), lambda i: (0, 0)),
            pl.BlockSpec((co, c_st), lambda i: (0, 0)),
            pl.BlockSpec((co, c_tx), lambda i: (0, 0)),
        ],
        out_specs=[
            pl.BlockSpec((co, _LB), lambda i: (0, i)),
            pl.BlockSpec((co, 1), lambda i: (0, 0)),
            pl.BlockSpec((co, 1), lambda i: (0, 0)),
        ],
        out_shape=[
            jax.ShapeDtypeStruct((co, nb * _LB), jnp.bfloat16),
            jax.ShapeDtypeStruct((co, 1), jnp.float32),
            jax.ShapeDtypeStruct((co, 1), jnp.float32),
        ],
    )(sp, st, tex, w_sp, w_st, w_tx)


def _norm_relu_cn(y, a, c, n):
    nb = -(-n // _LB)
    co = y.shape[0]

    def body(y_ref, a_ref, c_ref, o_ref):
        y = y_ref[...].astype(jnp.float32)
        o_ref[...] = jnp.maximum(a_ref[...] * y + c_ref[...], 0.0)

    return pl.pallas_call(
        body,
        grid=(nb,),
        in_specs=[
            pl.BlockSpec((co, _LB), lambda i: (0, i)),
            pl.BlockSpec((co, 1), lambda i: (0, 0)),
            pl.BlockSpec((co, 1), lambda i: (0, 0)),
        ],
        out_specs=pl.BlockSpec((co, _LB), lambda i: (0, i)),
        out_shape=jax.ShapeDtypeStruct((co, n), jnp.float32),
    )(y, a, c)


def _neighbor_conv_stats(st1d, gath1d, wbig, n2, npad):
    nb = -(-n2 // _RBP)
    kstride = npad * 64 // (_RBP * 128)
    blk = _RBP * 128

    def body(f_ref, g0_ref, g1_ref, g2_ref, w_ref, y_ref, s1_ref, s2_ref):
        i = pl.program_id(0)
        f = jnp.reshape(f_ref[...], (_RBP, 128))
        n0 = jnp.reshape(g0_ref[...], (_RBP, 128))
        n1 = jnp.reshape(g1_ref[...], (_RBP, 128))
        n2_ = jnp.reshape(g2_ref[...], (_RBP, 128))
        s_sum = n0 + n1 + n2_
        s_dif = jnp.abs(n2_ - n1) + 2.0 * jnp.abs(n1 - n0)
        s_div = jnp.abs(n0 - f) + jnp.abs(n1 - f) + jnp.abs(n2_ - f)
        z = jnp.concatenate([f, s_sum, s_dif, s_div], axis=1)
        y = lax.dot_general(z, w_ref[...], (((1,), (0,)), ((), ())),
                            preferred_element_type=jnp.float32,
                            precision=_PREC)
        y_ref[...] = y.astype(jnp.bfloat16)
        ids = i * _RBP + lax.broadcasted_iota(jnp.int32, y.shape, 0)
        ym = jnp.where(ids < n2, y, 0.0)

        @pl.when(i == 0)
        def _():
            s1_ref[...] = jnp.zeros_like(s1_ref)
            s2_ref[...] = jnp.zeros_like(s2_ref)

        s1_ref[...] += jnp.sum(ym, axis=0, keepdims=True)
        s2_ref[...] += jnp.sum(ym * ym, axis=0, keepdims=True)

    return pl.pallas_call(
        body,
        grid=(nb,),
        in_specs=[
            pl.BlockSpec((blk,), lambda i: (i,)),
            pl.BlockSpec((blk,), lambda i: (i,)),
            pl.BlockSpec((blk,), lambda i: (i + kstride,)),
            pl.BlockSpec((blk,), lambda i: (i + 2 * kstride,)),
            pl.BlockSpec((512, 128), lambda i: (0, 0)),
        ],
        out_specs=[
            pl.BlockSpec((_RBP, 128), lambda i: (i, 0)),
            pl.BlockSpec((1, 128), lambda i: (0, 0)),
            pl.BlockSpec((1, 128), lambda i: (0, 0)),
        ],
        out_shape=[
            jax.ShapeDtypeStruct((nb * _RBP, 128), jnp.bfloat16),
            jax.ShapeDtypeStruct((1, 128), jnp.float32),
            jax.ShapeDtypeStruct((1, 128), jnp.float32),
        ],
    )(st1d, gath1d, gath1d, gath1d, wbig)


def _norm_relu_conv_stats(y2, a, c, wbd, n2):
    nb = -(-n2 // _RBP2)

    def body(y_ref, a_ref, c_ref, w_ref, y3_ref, s1_ref, s2_ref):
        i = pl.program_id(0)
        y2f = y_ref[...].astype(jnp.float32)
        st1 = jnp.maximum(a_ref[...] * y2f + c_ref[...], 0.0)
        y3 = lax.dot_general(st1, w_ref[...], (((1,), (0,)), ((), ())),
                             preferred_element_type=jnp.float32,
                             precision=_PREC)
        y3_ref[...] = y3.astype(jnp.bfloat16)
        ids = i * _RBP2 + lax.broadcasted_iota(jnp.int32, y3.shape, 0)
        ym = jnp.where(ids < n2, y3, 0.0)

        @pl.when(i == 0)
        def _():
            s1_ref[...] = jnp.zeros_like(s1_ref)
            s2_ref[...] = jnp.zeros_like(s2_ref)

        s1_ref[...] += jnp.sum(ym, axis=0, keepdims=True)
        s2_ref[...] += jnp.sum(ym * ym, axis=0, keepdims=True)

    return pl.pallas_call(
        body,
        grid=(nb,),
        in_specs=[
            pl.BlockSpec((_RBP2, 128), lambda i: (i, 0)),
            pl.BlockSpec((1, 128), lambda i: (0, 0)),
            pl.BlockSpec((1, 128), lambda i: (0, 0)),
            pl.BlockSpec((128, 128), lambda i: (0, 0)),
        ],
        out_specs=[
            pl.BlockSpec((_RBP2, 128), lambda i: (i, 0)),
            pl.BlockSpec((1, 128), lambda i: (0, 0)),
            pl.BlockSpec((1, 128), lambda i: (0, 0)),
        ],
        out_shape=[
            jax.ShapeDtypeStruct((nb * _RBP2, 128), jnp.bfloat16),
            jax.ShapeDtypeStruct((1, 128), jnp.float32),
            jax.ShapeDtypeStruct((1, 128), jnp.float32),
        ],
    )(y2, a, c, wbd)


def _bn_coefs(s1, s2, gamma, beta, n):
    m = s1 / n
    v = s2 / n - m * m
    a = gamma * lax.rsqrt(v + _EPS)
    return a, beta - a * m


def kernel(spatial_fea, structural_fea, tex_fea, neighbor_index,
           W1, b1, g1, be1, W2, b2, g2, be2, W3, b3, g3, be3):
    B, c_sp, N = spatial_fea.shape
    c_st = structural_fea.shape[1]
    del b1, b2, b3

    sp = spatial_fea[0]
    st = structural_fea[0]
    tex = tex_fea[0]

    unit = 32 * _GRP // 3 if (32 * _GRP) % 3 == 0 else 32 * _GRP
    npad = -(-N // unit) * unit
    st1d = st.T.reshape(-1)
    stT = st1d.reshape(N, 64)
    zpad = jnp.zeros((npad - N,), jnp.int32)
    nix = neighbor_index[0]
    idx_flat = jnp.concatenate(
        [nix[:, 0], zpad, nix[:, 1], zpad, nix[:, 2], zpad])
    gath = _gather_rows(stT, idx_flat)
    gath1d = gath.reshape(-1)

    y1, s1, q1 = _conv_stats_cn(sp, st, tex, W1[:, :c_sp],
                                W1[:, c_sp:c_sp + c_st],
                                W1[:, c_sp + c_st:], N)
    a1, c1 = _bn_coefs(s1[:, 0], q1[:, 0], g1, be1, N)
    out_sp = _norm_relu_cn(y1, a1.reshape(-1, 1), c1.reshape(-1, 1), N)

    gath1d, out_sp = lax.optimization_barrier((gath1d, out_sp))

    n2 = N // 2
    eye2 = jnp.eye(2, dtype=jnp.float32)
    W2T = W2.T
    wbig = jnp.concatenate(
        [jnp.kron(eye2, W2T[k * c_st:(k + 1) * c_st]) for k in range(4)],
        axis=0)
    y2, s2, q2 = _neighbor_conv_stats(st1d, gath1d, wbig, n2, npad)
    a2, c2 = _bn_coefs(s2[0, :64] + s2[0, 64:], q2[0, :64] + q2[0, 64:],
                       g2, be2, N)
    a2p = jnp.tile(a2, 2).reshape(1, 128)
    c2p = jnp.tile(c2, 2).reshape(1, 128)
    w3bd = jnp.kron(eye2, W3.T)
    y3, s3, q3 = _norm_relu_conv_stats(y2, a2p, c2p, w3bd, n2)
    a3, c3 = _bn_coefs(s3[0, :64] + s3[0, 64:], q3[0, :64] + q3[0, 64:],
                       g3, be3, N)
    y3t = y3[:n2].reshape(N, 64).T
    out_st = _norm_relu_cn(y3t, a3.reshape(-1, 1), c3.reshape(-1, 1), N)

    return out_sp[None], out_st[None]
